# Initial kernel scaffold; baseline (speedup 1.0000x reference)
import jax
import jax.numpy as jnp
from jax.experimental import pallas as pl


def kernel(s, v, edge_index, edge_d, edge_vec, gamma0, beta0, Wrbf0, W1_0, b1_0, W2_0, b2_0, Wv0, Wu1_0, bu1_0, Wu2_0, bu2_0, gamma1, beta1, Wrbf1, W1_1, b1_1, W2_1, b2_1, Wv1, Wu1_1, bu1_1, Wu2_1, bu2_1, gamma2, beta2, Wrbf2, W1_2, b1_2, W2_2, b2_2, Wv2, Wu1_2, bu1_2, Wu2_2, bu2_2, gamma3, beta3, Wrbf3, W1_3, b1_3, W2_3, b2_3, Wv3, Wu1_3, bu1_3, Wu2_3, bu2_3, gamma4, beta4, Wrbf4, W1_4, b1_4, W2_4, b2_4, Wv4):
    raise NotImplementedError("write your pallas kernel here")


# speedup vs baseline: 8.9282x; 8.9282x over previous
import functools
import jax
import jax.numpy as jnp
import numpy as np
from jax.experimental import pallas as pl
from jax.experimental.pallas import tpu as pltpu

_S = 64
_V = 16
_R = 64
_CUTOFF = 10.0
_L = 5
_EB = 2000


def _edge_body(l, *refs):
    if l > 0:
        (sd_ref, ss_ref, vs_ref, d_ref, a_ref,
         wrbf_ref, w1_ref, b1_ref, w2_ref, b2_ref, ms_ref, vm_ref) = refs
    else:
        (sd_ref, ss_ref, d_ref, a_ref,
         wrbf_ref, w1_ref, b1_ref, w2_ref, b2_ref, ms_ref, vm_ref) = refs
        vs_ref = None
    d = d_ref[:, 0:1]
    centers = jax.lax.broadcasted_iota(jnp.int32, (1, _R), 1).astype(jnp.float32) * (_CUTOFF / (_R - 1))
    width = _CUTOFF / _R
    rbf = jnp.exp(-0.5 * ((d - centers) / width) ** 2)
    env = 0.5 * (jnp.cos(jnp.pi * jnp.clip(d, 0.0, _CUTOFF) / _CUTOFF) + 1.0)
    filt = jnp.dot(rbf, wrbf_ref[...], preferred_element_type=jnp.float32)
    sd = sd_ref[...]
    ss = ss_ref[...]
    a0 = a_ref[:, 0:1]
    a1 = a_ref[:, 1:2]
    a2 = a_ref[:, 2:3]
    if l > 0:
        vs = vs_ref[...]
        vdot = vs[:, 0:16] * a0 + vs[:, 16:32] * a1 + vs[:, 32:48] * a2
        m_in = jnp.concatenate([sd, ss, vdot], axis=1)
    else:
        m_in = jnp.concatenate([sd, ss], axis=1)
    h = m_in @ w1_ref[...] + b1_ref[...]
    h = h * jax.nn.sigmoid(h)
    h = h * filt
    o = h @ w2_ref[...] + b2_ref[...]
    ms_ref[...] = o[:, :_S] * env
    g0 = o[:, _S:_S + _V]
    g1 = o[:, _S + _V:]
    if l > 0:
        vm = jnp.concatenate([g0 * a0 + g1 * vs[:, 0:16],
                              g0 * a1 + g1 * vs[:, 16:32],
                              g0 * a2 + g1 * vs[:, 32:48]], axis=1)
    else:
        vm = jnp.concatenate([g0 * a0, g0 * a1, g0 * a2], axis=1)
    vm_ref[...] = vm * env


def _edge_mlp(l, sd, ss, vs, d2, a, wrbf, w1, b1, w2, b2, interpret=False):
    E = sd.shape[0]
    eb = _EB if E % _EB == 0 else E
    grid = (E // eb,)
    din = 2 * _S + (_V if l > 0 else 0)
    bspec = lambda w: pl.BlockSpec((eb, w), lambda i: (i, 0))
    wspec = lambda r, c: pl.BlockSpec((r, c), lambda i: (0, 0))
    in_specs = [bspec(_S), bspec(_S)]
    args = [sd, ss]
    if l > 0:
        in_specs.append(bspec(3 * _V))
        args.append(vs)
    in_specs += [bspec(1), bspec(3),
                 wspec(_R, _S), wspec(din, _S), wspec(1, _S),
                 wspec(_S, _S + 2 * _V), wspec(1, _S + 2 * _V)]
    args += [d2, a, wrbf, w1, b1.reshape(1, _S), w2, b2.reshape(1, _S + 2 * _V)]
    out_specs = [bspec(_S), bspec(3 * _V)]
    out_shape = [jax.ShapeDtypeStruct((E, _S), jnp.float32),
                 jax.ShapeDtypeStruct((E, 3 * _V), jnp.float32)]
    ms, vm = pl.pallas_call(
        functools.partial(_edge_body, l),
        grid=grid,
        in_specs=in_specs,
        out_specs=out_specs,
        out_shape=out_shape,
        interpret=interpret,
    )(*args)
    return ms, vm


def _forward(P, s, v, d, a, edge_index, interpret=False):
    n = s.shape[0]
    src = edge_index[0]
    dst = edge_index[1]
    d2 = d[:, None]
    deg = jax.ops.segment_sum(jnp.ones_like(d), dst, num_segments=n)
    deg = jnp.maximum(deg, 1.0)
    vflat = v.reshape(n, 3 * _V)
    for l in range(_L):
        mu = jnp.mean(s, axis=-1, keepdims=True)
        var = jnp.var(s, axis=-1, keepdims=True)
        s = (s - mu) / jnp.sqrt(var + 1e-6) * P['gamma%d' % l] + P['beta%d' % l]
        vn = jnp.sqrt(jnp.mean(vflat * vflat, axis=1) + 1e-6)
        vflat = vflat / vn[:, None]
        sd = jnp.take(s, dst, axis=0)
        ss = jnp.take(s, src, axis=0)
        vs = jnp.take(vflat, src, axis=0) if l > 0 else None
        ms, vm = _edge_mlp(l, sd, ss, vs, d2, a,
                           P['Wrbf%d' % l], P['W1_%d' % l], P['b1_%d' % l],
                           P['W2_%d' % l], P['b2_%d' % l], interpret=interpret)
        s = s + jax.ops.segment_sum(ms, dst, num_segments=n)
        v_agg = jax.ops.segment_sum(vm, dst, num_segments=n) / deg[:, None]
        wv = P['Wv%d' % l]
        va = v_agg.reshape(n, 3, _V)
        vflat = vflat + (va @ wv).reshape(n, 3 * _V)
        if l < _L - 1:
            s = s + (jax.nn.silu(s @ P['Wu1_%d' % l] + P['bu1_%d' % l]) @ P['Wu2_%d' % l] + P['bu2_%d' % l])
    return s, vflat.reshape(n, 3, _V)


def kernel(s, v, edge_index, edge_d, edge_vec, gamma0, beta0, Wrbf0, W1_0, b1_0, W2_0, b2_0, Wv0, Wu1_0, bu1_0, Wu2_0, bu2_0, gamma1, beta1, Wrbf1, W1_1, b1_1, W2_1, b2_1, Wv1, Wu1_1, bu1_1, Wu2_1, bu2_1, gamma2, beta2, Wrbf2, W1_2, b1_2, W2_2, b2_2, Wv2, Wu1_2, bu1_2, Wu2_2, bu2_2, gamma3, beta3, Wrbf3, W1_3, b1_3, W2_3, b2_3, Wv3, Wu1_3, bu1_3, Wu2_3, bu2_3, gamma4, beta4, Wrbf4, W1_4, b1_4, W2_4, b2_4, Wv4):
    kw = dict(locals())
    edge_index = kw.pop('edge_index')
    s = kw.pop('s')
    v = kw.pop('v')
    d = kw.pop('edge_d')
    a = kw.pop('edge_vec')
    return _forward(kw, s, v, d, a, edge_index)


# speedup vs baseline: 18.8708x; 2.1136x over previous
import functools
import jax
import jax.numpy as jnp
import numpy as np
from jax import lax
from jax.experimental import pallas as pl
from jax.experimental.pallas import tpu as pltpu
from jax.experimental.pallas import tpu_sc as plsc

_S = 64
_V = 16
_R = 64
_CUTOFF = 10.0
_L = 5
_N = 50000
_E = 800000

_W = 128
_EB = 2048
_EPAD = 819200
_GW = 32
_GCH = 128
_GNCH = _EPAD // (_GW * _GCH)

_NB = 8
_BKT = 6250
_ACC = 6400
_PT = _ACC // 16
_CPB = 2048
_EL = _E + _NB * _CPB


def _edge_body(l, *refs):
    (gd_ref, gs_ref, d_ref, a_ref,
     wrbf_ref, w1_ref, b1_ref, w2_ref, b2_ref, out_ref) = refs
    d = d_ref[:, 0:1]
    centers = jax.lax.broadcasted_iota(jnp.int32, (1, _R), 1).astype(jnp.float32) * (_CUTOFF / (_R - 1))
    width = _CUTOFF / _R
    rbf = jnp.exp(-0.5 * ((d - centers) / width) ** 2)
    env = 0.5 * (jnp.cos(jnp.pi * jnp.clip(d, 0.0, _CUTOFF) / _CUTOFF) + 1.0)
    filt = jnp.dot(rbf, wrbf_ref[...], preferred_element_type=jnp.float32)
    sd = gd_ref[:, 0:_S]
    ss = gs_ref[:, 0:_S]
    a0 = a_ref[:, 0:1]
    a1 = a_ref[:, 1:2]
    a2 = a_ref[:, 2:3]
    if l > 0:
        vs = gs_ref[:, _S:_S + 3 * _V]
        vdot = vs[:, 0:16] * a0 + vs[:, 16:32] * a1 + vs[:, 32:48] * a2
        m_in = jnp.concatenate([sd, ss, vdot], axis=1)
    else:
        m_in = jnp.concatenate([sd, ss], axis=1)
    h = m_in @ w1_ref[...] + b1_ref[...]
    h = h * jax.nn.sigmoid(h)
    h = h * filt
    o = h @ w2_ref[...] + b2_ref[...]
    ms = o[:, :_S] * env
    g0 = o[:, _S:_S + _V]
    g1 = o[:, _S + _V:]
    if l > 0:
        vm = jnp.concatenate([g0 * a0 + g1 * vs[:, 0:16],
                              g0 * a1 + g1 * vs[:, 16:32],
                              g0 * a2 + g1 * vs[:, 32:48]], axis=1)
    else:
        vm = jnp.concatenate([g0 * a0, g0 * a1, g0 * a2], axis=1)
    pad = jnp.zeros((_EB, _W - _S - 3 * _V), jnp.float32)
    out_ref[...] = jnp.concatenate([ms, vm * env, pad], axis=1)


def _edge_mlp(l, gd, gs, d2, a, wrbf, w1, b1, w2, b2):
    grid = (_EPAD // _EB,)
    din = 2 * _S + (_V if l > 0 else 0)
    bspec = lambda w: pl.BlockSpec((_EB, w), lambda i: (i, 0))
    wspec = lambda r, c: pl.BlockSpec((r, c), lambda i: (0, 0))
    in_specs = [bspec(_W), bspec(_W), bspec(1), bspec(3),
                wspec(_R, _S), wspec(din, _S), wspec(1, _S),
                wspec(_S, _S + 2 * _V), wspec(1, _S + 2 * _V)]
    args = [gd, gs, d2, a, wrbf, w1, b1.reshape(1, _S), w2, b2.reshape(1, _S + 2 * _V)]
    return pl.pallas_call(
        functools.partial(_edge_body, l),
        grid=grid,
        in_specs=in_specs,
        out_specs=pl.BlockSpec((_EB, _W), lambda i: (i, 0)),
        out_shape=jax.ShapeDtypeStruct((_EPAD, _W), jnp.float32),
    )(*args)


def _sc_mesh():
    return plsc.VectorSubcoreMesh(core_axis_name="c", subcore_axis_name="s")


def _gather_body(tab, srci, dsti, gd, gs, idx_s, idx_d, bufd, bufs, sem):
    wid = lax.axis_index("s") * 2 + lax.axis_index("c")
    base = wid * (_GNCH * _GCH)

    def body(j, carry):
        st = base + j * _GCH
        pltpu.sync_copy(srci.at[pl.ds(st, _GCH)], idx_s)
        pltpu.sync_copy(dsti.at[pl.ds(st, _GCH)], idx_d)
        c1 = pltpu.async_copy(tab.at[idx_d], bufd, sem)
        c2 = pltpu.async_copy(tab.at[idx_s], bufs, sem)
        c1.wait()
        c2.wait()
        pltpu.sync_copy(bufd, gd.at[pl.ds(st, _GCH)])
        pltpu.sync_copy(bufs, gs.at[pl.ds(st, _GCH)])
        return carry

    lax.fori_loop(0, _GNCH, body, 0)


def _sc_gather(tab, srci, dsti):
    return pl.kernel(
        _gather_body,
        out_type=[jax.ShapeDtypeStruct((_EPAD, _W), jnp.float32),
                  jax.ShapeDtypeStruct((_EPAD, _W), jnp.float32)],
        mesh=_sc_mesh(),
        scratch_types=[
            pltpu.VMEM((_GCH,), jnp.int32),
            pltpu.VMEM((_GCH,), jnp.int32),
            pltpu.VMEM((_GCH, _W), jnp.float32),
            pltpu.VMEM((_GCH, _W), jnp.float32),
            pltpu.SemaphoreType.DMA,
        ],
    )(tab, srci, dsti)


def _scatter_body(pay, eids, ldst, offs, zrow, out, offv, eidb, ldb, pbuf, zbuf, acc, sem):
    c = lax.axis_index("c")
    s = lax.axis_index("s")
    pltpu.sync_copy(offs, offv)
    pltpu.sync_copy(zrow, zbuf)
    ov = offv[...]
    oly = [ov[i] for i in range(_NB + 1)]

    nfull = _PT // _GCH
    rem = _PT % _GCH
    for bb in range(_NB // 2):
        b = c * (_NB // 2) + bb
        off_b = lax.select(c == 0, oly[bb], oly[_NB // 2 + bb])
        off_b1 = lax.select(c == 0, oly[bb + 1], oly[_NB // 2 + bb + 1])
        nch = (off_b1 - off_b) // _GCH
        nj = (nch + 15 - s) // 16
        for k in range(nfull):
            pltpu.sync_copy(zbuf, acc.at[pl.ds(s * _PT + k * _GCH, _GCH)])
        if rem:
            pltpu.sync_copy(zbuf.at[pl.ds(0, rem)],
                            acc.at[pl.ds(s * _PT + nfull * _GCH, rem)])
        plsc.subcore_barrier()

        def body(j, carry):
            st = pl.multiple_of(off_b + (j * 16 + s) * _GCH, _GCH)
            pltpu.sync_copy(eids.at[pl.ds(st, _GCH)], eidb)
            pltpu.sync_copy(ldst.at[pl.ds(st, _GCH)], ldb.at[0])
            pltpu.async_copy(pay.at[eidb], pbuf, sem).wait()
            pltpu.sync_copy(pbuf, acc.at[ldb.at[0]], add=True)
            return carry

        lax.fori_loop(0, nj, body, 0)
        plsc.subcore_barrier()
        ob = b * _ACC + s * _PT
        for k in range(nfull):
            pltpu.sync_copy(acc.at[pl.ds(s * _PT + k * _GCH, _GCH)], pbuf)
            pltpu.sync_copy(pbuf, out.at[pl.ds(ob + k * _GCH, _GCH)])
        if rem:
            pltpu.sync_copy(acc.at[pl.ds(s * _PT + nfull * _GCH, rem)],
                            pbuf.at[pl.ds(0, rem)])
            pltpu.sync_copy(pbuf.at[pl.ds(0, rem)],
                            out.at[pl.ds(ob + nfull * _GCH, rem)])
        plsc.subcore_barrier()


def _sc_scatter(pay, eids, ldst, offs, zrow):
    return pl.kernel(
        _scatter_body,
        out_type=jax.ShapeDtypeStruct((_NB * _ACC, _W), jnp.float32),
        mesh=_sc_mesh(),
        scratch_types=[
            pltpu.VMEM((16,), jnp.int32),
            pltpu.VMEM((_GCH,), jnp.int32),
            pltpu.VMEM((1, _GCH), jnp.int32),
            pltpu.VMEM((_GCH, _W), jnp.float32),
            pltpu.VMEM((_GCH, _W), jnp.float32),
            pltpu.VMEM_SHARED((_ACC, _W), jnp.float32),
            pltpu.SemaphoreType.DMA,
        ],
    )(pay, eids, ldst, offs, zrow)


def _build_buckets(dst):
    e_iota = jnp.arange(_E, dtype=jnp.int32)
    bucket = dst // _BKT
    sb, perm = lax.sort_key_val(bucket, e_iota)
    qs5 = jnp.arange(_NB + 1, dtype=jnp.int32)
    off_c = jnp.sum(sb[None, :] < qs5[:, None], axis=1).astype(jnp.int32)
    cnt = off_c[1:] - off_c[:-1]
    cnt_pad = ((cnt + _CPB - 1) // _CPB) * _CPB
    off_pad = jnp.concatenate([jnp.zeros((1,), jnp.int32), jnp.cumsum(cnt_pad).astype(jnp.int32)])
    qs = jnp.arange(_EL, dtype=jnp.int32)
    bq = jnp.sum(qs[:, None] >= off_pad[None, 1:_NB], axis=1).astype(jnp.int32)
    rank = qs - off_pad[bq]
    valid = rank < cnt[bq]
    srci = jnp.clip(off_c[bq] + rank, 0, _E - 1)
    eids = jnp.where(valid, perm[srci], qs % _E)
    dstp = dst[perm]
    ldst = jnp.where(valid, dstp[srci] - _BKT * bq, _BKT + (qs % (_ACC - _BKT)))
    offs = jnp.zeros((16,), jnp.int32).at[:_NB + 1].set(off_pad)
    return eids, ldst, offs


def _forward(P, s, v, d, a, edge_index):
    n = s.shape[0]
    src = edge_index[0]
    dst = edge_index[1]
    eids, ldst, offs = _build_buckets(dst)
    zrow = jnp.zeros((_GCH, _W), jnp.float32)
    padi = (jnp.arange(_EPAD - _E, dtype=jnp.int32) % _N)
    src_p = jnp.concatenate([src, padi])
    dst_p = jnp.concatenate([dst, padi])
    d2 = jnp.concatenate([d, jnp.zeros((_EPAD - _E,), jnp.float32)])[:, None]
    a_p = jnp.concatenate([a, jnp.zeros((_EPAD - _E, 3), jnp.float32)], axis=0)

    deg = jax.ops.segment_sum(jnp.ones((_E,), jnp.float32), dst, num_segments=n)
    deg = jnp.maximum(deg, 1.0)
    vflat = v.reshape(n, 3 * _V)
    zpad = jnp.zeros((n, _W - _S - 3 * _V), jnp.float32)
    for l in range(_L):
        mu = jnp.mean(s, axis=-1, keepdims=True)
        var = jnp.var(s, axis=-1, keepdims=True)
        s = (s - mu) / jnp.sqrt(var + 1e-6) * P['gamma%d' % l] + P['beta%d' % l]
        vn = jnp.sqrt(jnp.mean(vflat * vflat, axis=1) + 1e-6)
        vflat = vflat / vn[:, None]
        tab = jnp.concatenate([s, vflat, zpad], axis=1)
        gd, gs = _sc_gather(tab, src_p, dst_p)
        pay = _edge_mlp(l, gd, gs, d2, a_p,
                        P['Wrbf%d' % l], P['W1_%d' % l], P['b1_%d' % l],
                        P['W2_%d' % l], P['b2_%d' % l])
        agg = _sc_scatter(pay, eids, ldst, offs, zrow)
        agg = agg.reshape(_NB, _ACC, _W)[:, :_BKT, :].reshape(n, _W)
        s = s + agg[:, :_S]
        v_agg = agg[:, _S:_S + 3 * _V] / deg[:, None]
        wv = P['Wv%d' % l]
        va = v_agg.reshape(n, 3, _V)
        vflat = vflat + (va @ wv).reshape(n, 3 * _V)
        if l < _L - 1:
            s = s + (jax.nn.silu(s @ P['Wu1_%d' % l] + P['bu1_%d' % l]) @ P['Wu2_%d' % l] + P['bu2_%d' % l])
    return s, vflat.reshape(n, 3, _V)


def kernel(s, v, edge_index, edge_d, edge_vec, gamma0, beta0, Wrbf0, W1_0, b1_0, W2_0, b2_0, Wv0, Wu1_0, bu1_0, Wu2_0, bu2_0, gamma1, beta1, Wrbf1, W1_1, b1_1, W2_1, b2_1, Wv1, Wu1_1, bu1_1, Wu2_1, bu2_1, gamma2, beta2, Wrbf2, W1_2, b1_2, W2_2, b2_2, Wv2, Wu1_2, bu1_2, Wu2_2, bu2_2, gamma3, beta3, Wrbf3, W1_3, b1_3, W2_3, b2_3, Wv3, Wu1_3, bu1_3, Wu2_3, bu2_3, gamma4, beta4, Wrbf4, W1_4, b1_4, W2_4, b2_4, Wv4):
    kw = dict(locals())
    edge_index = kw.pop('edge_index')
    s = kw.pop('s')
    v = kw.pop('v')
    d = kw.pop('edge_d')
    a = kw.pop('edge_vec')
    return _forward(kw, s, v, d, a, edge_index)


# speedup vs baseline: 18.8725x; 1.0001x over previous
import functools
import jax
import jax.numpy as jnp
import numpy as np
from jax import lax
from jax.experimental import pallas as pl
from jax.experimental.pallas import tpu as pltpu
from jax.experimental.pallas import tpu_sc as plsc

_S = 64
_V = 16
_R = 64
_CUTOFF = 10.0
_L = 5
_N = 50000
_E = 800000

_W = 128
_EB = 2048
_EPAD = 819200
_GW = 32
_GCH = 128
_GNCH = _EPAD // (_GW * _GCH)

_NB = 8
_BKT = 6250
_ACC = 6400
_PT = _ACC // 16
_CPB = 2048
_EL = _E + _NB * _CPB


def _edge_body(l, *refs):
    (gd_ref, gs_ref, d_ref, a_ref,
     wrbf_ref, w1_ref, b1_ref, w2_ref, b2_ref, out_ref) = refs
    d = d_ref[:, 0:1]
    centers = jax.lax.broadcasted_iota(jnp.int32, (1, _R), 1).astype(jnp.float32) * (_CUTOFF / (_R - 1))
    width = _CUTOFF / _R
    rbf = jnp.exp(-0.5 * ((d - centers) / width) ** 2)
    env = 0.5 * (jnp.cos(jnp.pi * jnp.clip(d, 0.0, _CUTOFF) / _CUTOFF) + 1.0)
    filt = jnp.dot(rbf, wrbf_ref[...], preferred_element_type=jnp.float32)
    sd = gd_ref[:, 0:_S]
    ss = gs_ref[:, 0:_S]
    a0 = a_ref[:, 0:1]
    a1 = a_ref[:, 1:2]
    a2 = a_ref[:, 2:3]
    if l > 0:
        vs = gs_ref[:, _S:_S + 3 * _V]
        vdot = vs[:, 0:16] * a0 + vs[:, 16:32] * a1 + vs[:, 32:48] * a2
        m_in = jnp.concatenate([sd, ss, vdot], axis=1)
    else:
        m_in = jnp.concatenate([sd, ss], axis=1)
    h = m_in @ w1_ref[...] + b1_ref[...]
    h = h * jax.nn.sigmoid(h)
    h = h * filt
    o = h @ w2_ref[...] + b2_ref[...]
    ms = o[:, :_S] * env
    g0 = o[:, _S:_S + _V]
    g1 = o[:, _S + _V:]
    if l > 0:
        vm = jnp.concatenate([g0 * a0 + g1 * vs[:, 0:16],
                              g0 * a1 + g1 * vs[:, 16:32],
                              g0 * a2 + g1 * vs[:, 32:48]], axis=1)
    else:
        vm = jnp.concatenate([g0 * a0, g0 * a1, g0 * a2], axis=1)
    pad = jnp.zeros((_EB, _W - _S - 3 * _V), jnp.float32)
    out_ref[...] = jnp.concatenate([ms, vm * env, pad], axis=1)


def _edge_mlp(l, gd, gs, d2, a, wrbf, w1, b1, w2, b2):
    grid = (_EPAD // _EB,)
    din = 2 * _S + (_V if l > 0 else 0)
    bspec = lambda w: pl.BlockSpec((_EB, w), lambda i: (i, 0))
    wspec = lambda r, c: pl.BlockSpec((r, c), lambda i: (0, 0))
    in_specs = [bspec(_W), bspec(_W), bspec(1), bspec(3),
                wspec(_R, _S), wspec(din, _S), wspec(1, _S),
                wspec(_S, _S + 2 * _V), wspec(1, _S + 2 * _V)]
    args = [gd, gs, d2, a, wrbf, w1, b1.reshape(1, _S), w2, b2.reshape(1, _S + 2 * _V)]
    return pl.pallas_call(
        functools.partial(_edge_body, l),
        grid=grid,
        in_specs=in_specs,
        out_specs=pl.BlockSpec((_EB, _W), lambda i: (i, 0)),
        out_shape=jax.ShapeDtypeStruct((_EPAD, _W), jnp.float32),
    )(*args)


def _sc_mesh():
    return plsc.VectorSubcoreMesh(core_axis_name="c", subcore_axis_name="s")


def _gather_body(tab, srci, dsti, gd, gs, idx_s, idx_d, bufd, bufs, sem):
    wid = lax.axis_index("s") * 2 + lax.axis_index("c")
    base = wid * (_GNCH * _GCH)

    def body(j, carry):
        st = base + j * _GCH
        pltpu.sync_copy(srci.at[pl.ds(st, _GCH)], idx_s)
        pltpu.sync_copy(dsti.at[pl.ds(st, _GCH)], idx_d)
        c1 = pltpu.async_copy(tab.at[idx_d], bufd, sem)
        c2 = pltpu.async_copy(tab.at[idx_s], bufs, sem)
        c1.wait()
        c2.wait()
        pltpu.sync_copy(bufd, gd.at[pl.ds(st, _GCH)])
        pltpu.sync_copy(bufs, gs.at[pl.ds(st, _GCH)])
        return carry

    lax.fori_loop(0, _GNCH, body, 0)


def _sc_gather(tab, srci, dsti):
    return pl.kernel(
        _gather_body,
        out_type=[jax.ShapeDtypeStruct((_EPAD, _W), jnp.float32),
                  jax.ShapeDtypeStruct((_EPAD, _W), jnp.float32)],
        mesh=_sc_mesh(),
        compiler_params=pltpu.CompilerParams(use_tc_tiling_on_sc=True),
        scratch_types=[
            pltpu.VMEM((_GCH,), jnp.int32),
            pltpu.VMEM((_GCH,), jnp.int32),
            pltpu.VMEM((_GCH, _W), jnp.float32),
            pltpu.VMEM((_GCH, _W), jnp.float32),
            pltpu.SemaphoreType.DMA,
        ],
    )(tab, srci, dsti)


def _scatter_body(pay, eids, ldst, offs, zrow, out, offv, eidb, ldb, pbuf, zbuf, acc, sem):
    c = lax.axis_index("c")
    s = lax.axis_index("s")
    pltpu.sync_copy(offs, offv)
    pltpu.sync_copy(zrow, zbuf)
    ov = offv[...]
    oly = [ov[i] for i in range(_NB + 1)]

    nfull = _PT // _GCH
    rem = _PT % _GCH
    for bb in range(_NB // 2):
        b = c * (_NB // 2) + bb
        off_b = lax.select(c == 0, oly[bb], oly[_NB // 2 + bb])
        off_b1 = lax.select(c == 0, oly[bb + 1], oly[_NB // 2 + bb + 1])
        nch = (off_b1 - off_b) // _GCH
        nj = (nch + 15 - s) // 16
        for k in range(nfull):
            pltpu.sync_copy(zbuf, acc.at[pl.ds(s * _PT + k * _GCH, _GCH)])
        if rem:
            pltpu.sync_copy(zbuf.at[pl.ds(0, rem)],
                            acc.at[pl.ds(s * _PT + nfull * _GCH, rem)])
        plsc.subcore_barrier()

        def body(j, carry):
            st = pl.multiple_of(off_b + (j * 16 + s) * _GCH, _GCH)
            pltpu.sync_copy(eids.at[pl.ds(st, _GCH)], eidb)
            pltpu.sync_copy(ldst.at[pl.ds(st, _GCH)], ldb.at[0])
            pltpu.async_copy(pay.at[eidb], pbuf, sem).wait()
            pltpu.sync_copy(pbuf, acc.at[ldb.at[0]], add=True)
            return carry

        lax.fori_loop(0, nj, body, 0)
        plsc.subcore_barrier()
        ob = b * _ACC + s * _PT
        for k in range(nfull):
            pltpu.sync_copy(acc.at[pl.ds(s * _PT + k * _GCH, _GCH)], pbuf)
            pltpu.sync_copy(pbuf, out.at[pl.ds(ob + k * _GCH, _GCH)])
        if rem:
            pltpu.sync_copy(acc.at[pl.ds(s * _PT + nfull * _GCH, rem)],
                            pbuf.at[pl.ds(0, rem)])
            pltpu.sync_copy(pbuf.at[pl.ds(0, rem)],
                            out.at[pl.ds(ob + nfull * _GCH, rem)])
        plsc.subcore_barrier()


def _sc_scatter(pay, eids, ldst, offs, zrow):
    return pl.kernel(
        _scatter_body,
        out_type=jax.ShapeDtypeStruct((_NB * _ACC, _W), jnp.float32),
        mesh=_sc_mesh(),
        compiler_params=pltpu.CompilerParams(use_tc_tiling_on_sc=True),
        scratch_types=[
            pltpu.VMEM((16,), jnp.int32),
            pltpu.VMEM((_GCH,), jnp.int32),
            pltpu.VMEM((1, _GCH), jnp.int32),
            pltpu.VMEM((_GCH, _W), jnp.float32),
            pltpu.VMEM((_GCH, _W), jnp.float32),
            pltpu.VMEM_SHARED((_ACC, _W), jnp.float32),
            pltpu.SemaphoreType.DMA,
        ],
    )(pay, eids, ldst, offs, zrow)


def _build_buckets(dst):
    e_iota = jnp.arange(_E, dtype=jnp.int32)
    bucket = dst // _BKT
    sb, perm = lax.sort_key_val(bucket, e_iota)
    qs5 = jnp.arange(_NB + 1, dtype=jnp.int32)
    off_c = jnp.sum(sb[None, :] < qs5[:, None], axis=1).astype(jnp.int32)
    cnt = off_c[1:] - off_c[:-1]
    cnt_pad = ((cnt + _CPB - 1) // _CPB) * _CPB
    off_pad = jnp.concatenate([jnp.zeros((1,), jnp.int32), jnp.cumsum(cnt_pad).astype(jnp.int32)])
    qs = jnp.arange(_EL, dtype=jnp.int32)
    bq = jnp.sum(qs[:, None] >= off_pad[None, 1:_NB], axis=1).astype(jnp.int32)
    rank = qs - off_pad[bq]
    valid = rank < cnt[bq]
    srci = jnp.clip(off_c[bq] + rank, 0, _E - 1)
    eids = jnp.where(valid, perm[srci], qs % _E)
    dstp = dst[perm]
    ldst = jnp.where(valid, dstp[srci] - _BKT * bq, _BKT + (qs % (_ACC - _BKT)))
    offs = jnp.zeros((16,), jnp.int32).at[:_NB + 1].set(off_pad)
    return eids, ldst, offs


def _forward(P, s, v, d, a, edge_index):
    n = s.shape[0]
    src = edge_index[0]
    dst = edge_index[1]
    eids, ldst, offs = _build_buckets(dst)
    zrow = jnp.zeros((_GCH, _W), jnp.float32)
    padi = (jnp.arange(_EPAD - _E, dtype=jnp.int32) % _N)
    src_p = jnp.concatenate([src, padi])
    dst_p = jnp.concatenate([dst, padi])
    d2 = jnp.concatenate([d, jnp.zeros((_EPAD - _E,), jnp.float32)])[:, None]
    a_p = jnp.concatenate([a, jnp.zeros((_EPAD - _E, 3), jnp.float32)], axis=0)

    deg = jax.ops.segment_sum(jnp.ones((_E,), jnp.float32), dst, num_segments=n)
    deg = jnp.maximum(deg, 1.0)
    vflat = v.reshape(n, 3 * _V)
    zpad = jnp.zeros((n, _W - _S - 3 * _V), jnp.float32)
    for l in range(_L):
        mu = jnp.mean(s, axis=-1, keepdims=True)
        var = jnp.var(s, axis=-1, keepdims=True)
        s = (s - mu) / jnp.sqrt(var + 1e-6) * P['gamma%d' % l] + P['beta%d' % l]
        vn = jnp.sqrt(jnp.mean(vflat * vflat, axis=1) + 1e-6)
        vflat = vflat / vn[:, None]
        tab = jnp.concatenate([s, vflat, zpad], axis=1)
        gd, gs = _sc_gather(tab, src_p, dst_p)
        pay = _edge_mlp(l, gd, gs, d2, a_p,
                        P['Wrbf%d' % l], P['W1_%d' % l], P['b1_%d' % l],
                        P['W2_%d' % l], P['b2_%d' % l])
        agg = _sc_scatter(pay, eids, ldst, offs, zrow)
        agg = agg.reshape(_NB, _ACC, _W)[:, :_BKT, :].reshape(n, _W)
        s = s + agg[:, :_S]
        v_agg = agg[:, _S:_S + 3 * _V] / deg[:, None]
        wv = P['Wv%d' % l]
        va = v_agg.reshape(n, 3, _V)
        vflat = vflat + (va @ wv).reshape(n, 3 * _V)
        if l < _L - 1:
            s = s + (jax.nn.silu(s @ P['Wu1_%d' % l] + P['bu1_%d' % l]) @ P['Wu2_%d' % l] + P['bu2_%d' % l])
    return s, vflat.reshape(n, 3, _V)


def kernel(s, v, edge_index, edge_d, edge_vec, gamma0, beta0, Wrbf0, W1_0, b1_0, W2_0, b2_0, Wv0, Wu1_0, bu1_0, Wu2_0, bu2_0, gamma1, beta1, Wrbf1, W1_1, b1_1, W2_1, b2_1, Wv1, Wu1_1, bu1_1, Wu2_1, bu2_1, gamma2, beta2, Wrbf2, W1_2, b1_2, W2_2, b2_2, Wv2, Wu1_2, bu1_2, Wu2_2, bu2_2, gamma3, beta3, Wrbf3, W1_3, b1_3, W2_3, b2_3, Wv3, Wu1_3, bu1_3, Wu2_3, bu2_3, gamma4, beta4, Wrbf4, W1_4, b1_4, W2_4, b2_4, Wv4):
    kw = dict(locals())
    edge_index = kw.pop('edge_index')
    s = kw.pop('s')
    v = kw.pop('v')
    d = kw.pop('edge_d')
    a = kw.pop('edge_vec')
    return _forward(kw, s, v, d, a, edge_index)


# speedup vs baseline: 19.0253x; 1.0081x over previous
import functools
import jax
import jax.numpy as jnp
import numpy as np
from jax import lax
from jax.experimental import pallas as pl
from jax.experimental.pallas import tpu as pltpu
from jax.experimental.pallas import tpu_sc as plsc

_S = 64
_V = 16
_R = 64
_CUTOFF = 10.0
_L = 5
_N = 50000
_E = 800000

_W = 128
_EB = 2048
_EPAD = 819200
_GW = 32
_GCH = 128
_GNCH = _EPAD // (_GW * _GCH)

_NB = 8
_BKT = 6250
_ACC = 6400
_PT = _ACC // 16
_CPB = 2048
_EL = _E + _NB * _CPB


def _edge_body(l, *refs):
    (gd_ref, gs_ref, d_ref, a_ref,
     wrbf_ref, w1_ref, b1_ref, w2_ref, b2_ref, out_ref) = refs
    d = d_ref[:, 0:1]
    centers = jax.lax.broadcasted_iota(jnp.int32, (1, _R), 1).astype(jnp.float32) * (_CUTOFF / (_R - 1))
    width = _CUTOFF / _R
    rbf = jnp.exp(-0.5 * ((d - centers) / width) ** 2)
    env = 0.5 * (jnp.cos(jnp.pi * jnp.clip(d, 0.0, _CUTOFF) / _CUTOFF) + 1.0)
    filt = jnp.dot(rbf, wrbf_ref[...], preferred_element_type=jnp.float32)
    sd = gd_ref[:, 0:_S]
    ss = gs_ref[:, 0:_S]
    a0 = a_ref[:, 0:1]
    a1 = a_ref[:, 1:2]
    a2 = a_ref[:, 2:3]
    if l > 0:
        vs = gs_ref[:, _S:_S + 3 * _V]
        vdot = vs[:, 0:16] * a0 + vs[:, 16:32] * a1 + vs[:, 32:48] * a2
        m_in = jnp.concatenate([sd, ss, vdot], axis=1)
    else:
        m_in = jnp.concatenate([sd, ss], axis=1)
    h = m_in @ w1_ref[...] + b1_ref[...]
    h = h * jax.nn.sigmoid(h)
    h = h * filt
    o = h @ w2_ref[...] + b2_ref[...]
    ms = o[:, :_S] * env
    g0 = o[:, _S:_S + _V]
    g1 = o[:, _S + _V:]
    if l > 0:
        vm = jnp.concatenate([g0 * a0 + g1 * vs[:, 0:16],
                              g0 * a1 + g1 * vs[:, 16:32],
                              g0 * a2 + g1 * vs[:, 32:48]], axis=1)
    else:
        vm = jnp.concatenate([g0 * a0, g0 * a1, g0 * a2], axis=1)
    pad = jnp.zeros((_EB, _W - _S - 3 * _V), jnp.float32)
    out_ref[...] = jnp.concatenate([ms, vm * env, pad], axis=1)


def _edge_mlp(l, gd, gs, d2, a, wrbf, w1, b1, w2, b2):
    grid = (_EPAD // _EB,)
    din = 2 * _S + (_V if l > 0 else 0)
    bspec = lambda w: pl.BlockSpec((_EB, w), lambda i: (i, 0))
    wspec = lambda r, c: pl.BlockSpec((r, c), lambda i: (0, 0))
    in_specs = [bspec(_W), bspec(_W), bspec(1), bspec(3),
                wspec(_R, _S), wspec(din, _S), wspec(1, _S),
                wspec(_S, _S + 2 * _V), wspec(1, _S + 2 * _V)]
    args = [gd, gs, d2, a, wrbf, w1, b1.reshape(1, _S), w2, b2.reshape(1, _S + 2 * _V)]
    return pl.pallas_call(
        functools.partial(_edge_body, l),
        grid=grid,
        in_specs=in_specs,
        out_specs=pl.BlockSpec((_EB, _W), lambda i: (i, 0)),
        out_shape=jax.ShapeDtypeStruct((_EPAD, _W), jnp.float32),
    )(*args)


def _sc_mesh():
    return plsc.VectorSubcoreMesh(core_axis_name="c", subcore_axis_name="s")


def _gather_body(tab, srci, dsti, gd, gs, idx_s, idx_d, bufd, bufs, sem):
    wid = lax.axis_index("s") * 2 + lax.axis_index("c")
    base = wid * (_GNCH * _GCH)

    def body(j, carry):
        st = base + j * _GCH
        pltpu.sync_copy(srci.at[pl.ds(st, _GCH)], idx_s)
        pltpu.sync_copy(dsti.at[pl.ds(st, _GCH)], idx_d)
        c1 = pltpu.async_copy(tab.at[idx_d], bufd, sem)
        c2 = pltpu.async_copy(tab.at[idx_s], bufs, sem)
        c1.wait()
        c2.wait()
        pltpu.sync_copy(bufd, gd.at[pl.ds(st, _GCH)])
        pltpu.sync_copy(bufs, gs.at[pl.ds(st, _GCH)])
        return carry

    lax.fori_loop(0, _GNCH, body, 0)


def _sc_gather(tab, srci, dsti):
    return pl.kernel(
        _gather_body,
        out_type=[jax.ShapeDtypeStruct((_EPAD, _W), jnp.float32),
                  jax.ShapeDtypeStruct((_EPAD, _W), jnp.float32)],
        mesh=_sc_mesh(),
        compiler_params=pltpu.CompilerParams(use_tc_tiling_on_sc=True),
        scratch_types=[
            pltpu.VMEM((_GCH,), jnp.int32),
            pltpu.VMEM((_GCH,), jnp.int32),
            pltpu.VMEM((_GCH, _W), jnp.float32),
            pltpu.VMEM((_GCH, _W), jnp.float32),
            pltpu.SemaphoreType.DMA,
        ],
    )(tab, srci, dsti)


def _scatter_body(pay, eids, ldst, offs, zrow, out, offv, eidb, ldb, pbuf, zbuf, acc, sem):
    c = lax.axis_index("c")
    s = lax.axis_index("s")
    pltpu.sync_copy(offs, offv)
    pltpu.sync_copy(zrow, zbuf)
    ov = offv[...]
    oly = [ov[i] for i in range(_NB + 1)]

    nfull = _PT // _GCH
    rem = _PT % _GCH
    for bb in range(_NB // 2):
        b = c * (_NB // 2) + bb
        off_b = lax.select(c == 0, oly[bb], oly[_NB // 2 + bb])
        off_b1 = lax.select(c == 0, oly[bb + 1], oly[_NB // 2 + bb + 1])
        nch = (off_b1 - off_b) // _GCH
        nj = (nch + 15 - s) // 16
        for k in range(nfull):
            pltpu.sync_copy(zbuf, acc.at[pl.ds(s * _PT + k * _GCH, _GCH)])
        if rem:
            pltpu.sync_copy(zbuf.at[pl.ds(0, rem)],
                            acc.at[pl.ds(s * _PT + nfull * _GCH, rem)])
        plsc.subcore_barrier()

        def body(j, carry):
            st = pl.multiple_of(off_b + (j * 16 + s) * _GCH, _GCH)
            pltpu.sync_copy(eids.at[pl.ds(st, _GCH)], eidb)
            pltpu.sync_copy(ldst.at[pl.ds(st, _GCH)], ldb.at[0])
            pltpu.async_copy(pay.at[eidb], pbuf, sem).wait()
            pltpu.sync_copy(pbuf, acc.at[ldb.at[0]], add=True)
            return carry

        lax.fori_loop(0, nj, body, 0)
        plsc.subcore_barrier()
        ob = b * _ACC + s * _PT
        for k in range(nfull):
            pltpu.sync_copy(acc.at[pl.ds(s * _PT + k * _GCH, _GCH)], pbuf)
            pltpu.sync_copy(pbuf, out.at[pl.ds(ob + k * _GCH, _GCH)])
        if rem:
            pltpu.sync_copy(acc.at[pl.ds(s * _PT + nfull * _GCH, rem)],
                            pbuf.at[pl.ds(0, rem)])
            pltpu.sync_copy(pbuf.at[pl.ds(0, rem)],
                            out.at[pl.ds(ob + nfull * _GCH, rem)])
        plsc.subcore_barrier()


def _sc_scatter(pay, eids, ldst, offs, zrow):
    return pl.kernel(
        _scatter_body,
        out_type=jax.ShapeDtypeStruct((_NB * _ACC, _W), jnp.float32),
        mesh=_sc_mesh(),
        compiler_params=pltpu.CompilerParams(use_tc_tiling_on_sc=True),
        scratch_types=[
            pltpu.VMEM((16,), jnp.int32),
            pltpu.VMEM((_GCH,), jnp.int32),
            pltpu.VMEM((1, _GCH), jnp.int32),
            pltpu.VMEM((_GCH, _W), jnp.float32),
            pltpu.VMEM((_GCH, _W), jnp.float32),
            pltpu.VMEM_SHARED((_ACC, _W), jnp.float32),
            pltpu.SemaphoreType.DMA,
        ],
    )(pay, eids, ldst, offs, zrow)


def _build_buckets(dst):
    e_iota = jnp.arange(_E, dtype=jnp.int32)
    bucket = dst // _BKT
    sb, perm = lax.sort_key_val(bucket, e_iota)
    qs5 = jnp.arange(_NB + 1, dtype=jnp.int32)
    off_c = jnp.sum(sb[None, :] < qs5[:, None], axis=1).astype(jnp.int32)
    cnt = off_c[1:] - off_c[:-1]
    cnt_pad = ((cnt + _CPB - 1) // _CPB) * _CPB
    off_pad = jnp.concatenate([jnp.zeros((1,), jnp.int32), jnp.cumsum(cnt_pad).astype(jnp.int32)])
    qs = jnp.arange(_EL, dtype=jnp.int32)
    bq = jnp.sum(qs[:, None] >= off_pad[None, 1:_NB], axis=1).astype(jnp.int32)
    rank = qs - off_pad[bq]
    valid = rank < cnt[bq]
    srci = jnp.clip(off_c[bq] + rank, 0, _E - 1)
    eids = jnp.where(valid, perm[srci], qs % _E)
    dstp = dst[perm]
    ldst = jnp.where(valid, dstp[srci] - _BKT * bq, _BKT + (qs % (_ACC - _BKT)))
    offs = jnp.zeros((16,), jnp.int32).at[:_NB + 1].set(off_pad)
    return eids, ldst, offs


def _forward(P, s, v, d, a, edge_index):
    n = s.shape[0]
    src = edge_index[0]
    dst = edge_index[1]
    eids, ldst, offs = _build_buckets(dst)
    zrow = jnp.zeros((_GCH, _W), jnp.float32)
    padi = (jnp.arange(_EPAD - _E, dtype=jnp.int32) % _N)
    src_p = jnp.concatenate([src, padi])
    dst_p = jnp.concatenate([dst, padi])
    d2 = jnp.concatenate([d, jnp.zeros((_EPAD - _E,), jnp.float32)])[:, None]
    a_p = jnp.concatenate([a, jnp.zeros((_EPAD - _E, 3), jnp.float32)], axis=0)

    deg = jax.ops.segment_sum(jnp.ones((_E,), jnp.float32), dst, num_segments=n)
    deg = jnp.maximum(deg, 1.0)
    vflat = v.reshape(n, 3 * _V)
    zpad = jnp.zeros((n, _W - _S - 3 * _V), jnp.float32)
    for l in range(_L):
        mu = jnp.mean(s, axis=-1, keepdims=True)
        var = jnp.var(s, axis=-1, keepdims=True)
        s = (s - mu) / jnp.sqrt(var + 1e-6) * P['gamma%d' % l] + P['beta%d' % l]
        vn = jnp.sqrt(jnp.mean(vflat * vflat, axis=1) + 1e-6)
        vflat = vflat / vn[:, None]
        tab = jnp.concatenate([s, vflat, zpad], axis=1)
        gd, gs = _sc_gather(tab, src_p, dst_p)
        pay = _edge_mlp(l, gd, gs, d2, a_p,
                        P['Wrbf%d' % l], P['W1_%d' % l], P['b1_%d' % l],
                        P['W2_%d' % l], P['b2_%d' % l])
        agg = _sc_scatter(pay, eids, ldst, offs, zrow)
        agg = agg.reshape(_NB, _ACC, _W)[:, :_BKT, :].reshape(n, _W)
        s = s + agg[:, :_S]
        v_agg = agg[:, _S:_S + 3 * _V] / deg[:, None]
        wv3 = jnp.kron(jnp.eye(3, dtype=jnp.float32), P['Wv%d' % l])
        vflat = vflat + v_agg @ wv3
        if l < _L - 1:
            s = s + (jax.nn.silu(s @ P['Wu1_%d' % l] + P['bu1_%d' % l]) @ P['Wu2_%d' % l] + P['bu2_%d' % l])
    return s, vflat.reshape(n, 3, _V)


def kernel(s, v, edge_index, edge_d, edge_vec, gamma0, beta0, Wrbf0, W1_0, b1_0, W2_0, b2_0, Wv0, Wu1_0, bu1_0, Wu2_0, bu2_0, gamma1, beta1, Wrbf1, W1_1, b1_1, W2_1, b2_1, Wv1, Wu1_1, bu1_1, Wu2_1, bu2_1, gamma2, beta2, Wrbf2, W1_2, b1_2, W2_2, b2_2, Wv2, Wu1_2, bu1_2, Wu2_2, bu2_2, gamma3, beta3, Wrbf3, W1_3, b1_3, W2_3, b2_3, Wv3, Wu1_3, bu1_3, Wu2_3, bu2_3, gamma4, beta4, Wrbf4, W1_4, b1_4, W2_4, b2_4, Wv4):
    kw = dict(locals())
    edge_index = kw.pop('edge_index')
    s = kw.pop('s')
    v = kw.pop('v')
    d = kw.pop('edge_d')
    a = kw.pop('edge_vec')
    return _forward(kw, s, v, d, a, edge_index)


# speedup vs baseline: 19.3541x; 1.0173x over previous
import functools
import jax
import jax.numpy as jnp
import numpy as np
from jax import lax
from jax.experimental import pallas as pl
from jax.experimental.pallas import tpu as pltpu
from jax.experimental.pallas import tpu_sc as plsc

_S = 64
_V = 16
_R = 64
_CUTOFF = 10.0
_L = 5
_N = 50000
_E = 800000

_W = 128
_EB = 2048
_EPAD = 819200
_GW = 32
_GCH = 128
_GNCH = _EPAD // (_GW * _GCH)

_NB = 8
_BKT = 6250
_ACC = 6400
_PT = _ACC // 16
_CPB = 2048
_EL = _E + _NB * _CPB


def _edge_body(l, *refs):
    (gd_ref, gs_ref, d_ref, a_ref,
     wrbf_ref, w1_ref, b1_ref, w2_ref, b2_ref, out_ref) = refs
    d = d_ref[:, 0:1]
    centers = jax.lax.broadcasted_iota(jnp.int32, (1, _R), 1).astype(jnp.float32) * (_CUTOFF / (_R - 1))
    width = _CUTOFF / _R
    rbf = jnp.exp(-0.5 * ((d - centers) / width) ** 2)
    env = 0.5 * (jnp.cos(jnp.pi * jnp.clip(d, 0.0, _CUTOFF) / _CUTOFF) + 1.0)
    filt = jnp.dot(rbf, wrbf_ref[...], preferred_element_type=jnp.float32)
    sd = gd_ref[:, 0:_S]
    ss = gs_ref[:, 0:_S]
    a0 = a_ref[:, 0:1]
    a1 = a_ref[:, 1:2]
    a2 = a_ref[:, 2:3]
    if l > 0:
        vs = gs_ref[:, _S:_S + 3 * _V]
        vdot = vs[:, 0:16] * a0 + vs[:, 16:32] * a1 + vs[:, 32:48] * a2
        m_in = jnp.concatenate([sd, ss, vdot], axis=1)
    else:
        m_in = jnp.concatenate([sd, ss], axis=1)
    h = m_in @ w1_ref[...] + b1_ref[...]
    h = h * jax.nn.sigmoid(h)
    h = h * filt
    o = h @ w2_ref[...] + b2_ref[...]
    ms = o[:, :_S] * env
    g0 = o[:, _S:_S + _V]
    g1 = o[:, _S + _V:]
    if l > 0:
        vm = jnp.concatenate([g0 * a0 + g1 * vs[:, 0:16],
                              g0 * a1 + g1 * vs[:, 16:32],
                              g0 * a2 + g1 * vs[:, 32:48]], axis=1)
    else:
        vm = jnp.concatenate([g0 * a0, g0 * a1, g0 * a2], axis=1)
    pad = jnp.zeros((_EB, _W - _S - 3 * _V), jnp.float32)
    out_ref[...] = jnp.concatenate([ms, vm * env, pad], axis=1)


def _edge_mlp(l, gd, gs, d2, a, wrbf, w1, b1, w2, b2):
    grid = (_EPAD // _EB,)
    din = 2 * _S + (_V if l > 0 else 0)
    bspec = lambda w: pl.BlockSpec((_EB, w), lambda i: (i, 0))
    wspec = lambda r, c: pl.BlockSpec((r, c), lambda i: (0, 0))
    in_specs = [bspec(_W), bspec(_W), bspec(1), bspec(3),
                wspec(_R, _S), wspec(din, _S), wspec(1, _S),
                wspec(_S, _S + 2 * _V), wspec(1, _S + 2 * _V)]
    args = [gd, gs, d2, a, wrbf, w1, b1.reshape(1, _S), w2, b2.reshape(1, _S + 2 * _V)]
    return pl.pallas_call(
        functools.partial(_edge_body, l),
        grid=grid,
        in_specs=in_specs,
        out_specs=pl.BlockSpec((_EB, _W), lambda i: (i, 0)),
        out_shape=jax.ShapeDtypeStruct((_EPAD, _W), jnp.float32),
    )(*args)


def _sc_mesh():
    return plsc.VectorSubcoreMesh(core_axis_name="c", subcore_axis_name="s")


def _gather_body(tab, srci, dsti, gd, gs, idx_s, idx_d, bufd, bufs, sem):
    wid = lax.axis_index("s") * 2 + lax.axis_index("c")
    base = wid * (_GNCH * _GCH)

    def body(j, carry):
        st = base + j * _GCH
        pltpu.sync_copy(srci.at[pl.ds(st, _GCH)], idx_s)
        pltpu.sync_copy(dsti.at[pl.ds(st, _GCH)], idx_d)
        c1 = pltpu.async_copy(tab.at[idx_d], bufd, sem)
        c2 = pltpu.async_copy(tab.at[idx_s], bufs, sem)
        c1.wait()
        c2.wait()
        pltpu.sync_copy(bufd, gd.at[pl.ds(st, _GCH)])
        pltpu.sync_copy(bufs, gs.at[pl.ds(st, _GCH)])
        return carry

    lax.fori_loop(0, _GNCH, body, 0)


def _sc_gather(tab, srci, dsti):
    return pl.kernel(
        _gather_body,
        out_type=[jax.ShapeDtypeStruct((_EPAD, _W), jnp.float32),
                  jax.ShapeDtypeStruct((_EPAD, _W), jnp.float32)],
        mesh=_sc_mesh(),
        compiler_params=pltpu.CompilerParams(use_tc_tiling_on_sc=True),
        scratch_types=[
            pltpu.VMEM((_GCH,), jnp.int32),
            pltpu.VMEM((_GCH,), jnp.int32),
            pltpu.VMEM((_GCH, _W), jnp.float32),
            pltpu.VMEM((_GCH, _W), jnp.float32),
            pltpu.SemaphoreType.DMA,
        ],
    )(tab, srci, dsti)


def _scatter_body(pay, eids, ldst, offs, zrow, out, offv, eidb, ldb, pbuf, zbuf, acc, sem):
    c = lax.axis_index("c")
    s = lax.axis_index("s")
    pltpu.sync_copy(offs, offv)
    pltpu.sync_copy(zrow, zbuf)
    ov = offv[...]
    oly = [ov[i] for i in range(_NB + 1)]

    nfull = _PT // _GCH
    rem = _PT % _GCH
    for bb in range(_NB // 2):
        b = c * (_NB // 2) + bb
        off_b = lax.select(c == 0, oly[bb], oly[_NB // 2 + bb])
        off_b1 = lax.select(c == 0, oly[bb + 1], oly[_NB // 2 + bb + 1])
        nch = (off_b1 - off_b) // _GCH
        nj = (nch + 15 - s) // 16
        for k in range(nfull):
            pltpu.sync_copy(zbuf, acc.at[pl.ds(s * _PT + k * _GCH, _GCH)])
        if rem:
            pltpu.sync_copy(zbuf.at[pl.ds(0, rem)],
                            acc.at[pl.ds(s * _PT + nfull * _GCH, rem)])
        plsc.subcore_barrier()

        def body(j, carry):
            st = pl.multiple_of(off_b + (j * 16 + s) * _GCH, _GCH)
            pltpu.sync_copy(eids.at[pl.ds(st, _GCH)], eidb)
            pltpu.sync_copy(ldst.at[pl.ds(st, _GCH)], ldb.at[0])
            pltpu.async_copy(pay.at[eidb], pbuf, sem).wait()
            pltpu.sync_copy(pbuf, acc.at[ldb.at[0]], add=True)
            return carry

        lax.fori_loop(0, nj, body, 0)
        plsc.subcore_barrier()
        ob = b * _ACC + s * _PT
        for k in range(nfull):
            pltpu.sync_copy(acc.at[pl.ds(s * _PT + k * _GCH, _GCH)], pbuf)
            pltpu.sync_copy(pbuf, out.at[pl.ds(ob + k * _GCH, _GCH)])
        if rem:
            pltpu.sync_copy(acc.at[pl.ds(s * _PT + nfull * _GCH, rem)],
                            pbuf.at[pl.ds(0, rem)])
            pltpu.sync_copy(pbuf.at[pl.ds(0, rem)],
                            out.at[pl.ds(ob + nfull * _GCH, rem)])
        plsc.subcore_barrier()


def _sc_scatter(pay, eids, ldst, offs, zrow):
    return pl.kernel(
        _scatter_body,
        out_type=jax.ShapeDtypeStruct((_NB * _ACC, _W), jnp.float32),
        mesh=_sc_mesh(),
        compiler_params=pltpu.CompilerParams(use_tc_tiling_on_sc=True),
        scratch_types=[
            pltpu.VMEM((16,), jnp.int32),
            pltpu.VMEM((_GCH,), jnp.int32),
            pltpu.VMEM((1, _GCH), jnp.int32),
            pltpu.VMEM((_GCH, _W), jnp.float32),
            pltpu.VMEM((_GCH, _W), jnp.float32),
            pltpu.VMEM_SHARED((_ACC, _W), jnp.float32),
            pltpu.SemaphoreType.DMA,
        ],
    )(pay, eids, ldst, offs, zrow)


_NBLK = 2000


def _norm_tab(s1, v1, g_ref, b_ref):
    mu = jnp.mean(s1, axis=-1, keepdims=True)
    var = jnp.mean((s1 - mu) ** 2, axis=-1, keepdims=True)
    sn = (s1 - mu) / jnp.sqrt(var + 1e-6) * g_ref[...] + b_ref[...]
    vn = jnp.sqrt(jnp.mean(v1 * v1, axis=1, keepdims=True) + 1e-6)
    vnf = v1 / vn
    pad = jnp.zeros((s1.shape[0], _W - _S - 3 * _V), jnp.float32)
    return jnp.concatenate([sn, vnf, pad], axis=1)


def _node_pre_body(s_ref, v_ref, g_ref, b_ref, tab_ref):
    tab_ref[...] = _norm_tab(s_ref[...], v_ref[...], g_ref, b_ref)


def _node_pre(s, vflat, gamma, beta):
    grid = (_N // _NBLK,)
    return pl.pallas_call(
        _node_pre_body,
        grid=grid,
        in_specs=[pl.BlockSpec((_NBLK, _S), lambda i: (i, 0)),
                  pl.BlockSpec((_NBLK, 3 * _V), lambda i: (i, 0)),
                  pl.BlockSpec((1, _S), lambda i: (0, 0)),
                  pl.BlockSpec((1, _S), lambda i: (0, 0))],
        out_specs=pl.BlockSpec((_NBLK, _W), lambda i: (i, 0)),
        out_shape=jax.ShapeDtypeStruct((_N, _W), jnp.float32),
    )(s, vflat, gamma.reshape(1, _S), beta.reshape(1, _S))


def _node_upd_body(last, *refs):
    if last:
        (tab_ref, agg_ref, rdeg_ref, wv3_ref, s_out, v_out) = refs
    else:
        (tab_ref, agg_ref, rdeg_ref, wv3_ref,
         wu1_ref, bu1_ref, wu2_ref, bu2_ref, g_ref, b_ref, tab_out) = refs
    tab = tab_ref[...]
    agg = agg_ref[...]
    rdeg = rdeg_ref[...]
    s1 = tab[:, 0:_S] + agg[:, 0:_S]
    va = agg[:, _S:_S + 3 * _V] * rdeg
    v1 = tab[:, _S:_S + 3 * _V] + jnp.dot(va, wv3_ref[...], preferred_element_type=jnp.float32)
    if last:
        s_out[...] = s1
        v_out[...] = v1
    else:
        h = s1 @ wu1_ref[...] + bu1_ref[...]
        h = h * jax.nn.sigmoid(h)
        s1 = s1 + h @ wu2_ref[...] + bu2_ref[...]
        tab_out[...] = _norm_tab(s1, v1, g_ref, b_ref)


def _node_upd(tab, agg, rdeg, wv3, wu1, bu1, wu2, bu2, gnext, bnext):
    last = wu1 is None
    grid = (_N // _NBLK,)
    bspec = lambda w: pl.BlockSpec((_NBLK, w), lambda i: (i, 0))
    wspec = lambda r, c: pl.BlockSpec((r, c), lambda i: (0, 0))
    in_specs = [bspec(_W), bspec(_W), bspec(1), wspec(3 * _V, 3 * _V)]
    args = [tab, agg, rdeg, wv3]
    if last:
        out_specs = [bspec(_S), bspec(3 * _V)]
        out_shape = [jax.ShapeDtypeStruct((_N, _S), jnp.float32),
                     jax.ShapeDtypeStruct((_N, 3 * _V), jnp.float32)]
    else:
        in_specs += [wspec(_S, _S), wspec(1, _S), wspec(_S, _S), wspec(1, _S),
                     wspec(1, _S), wspec(1, _S)]
        args += [wu1, bu1.reshape(1, _S), wu2, bu2.reshape(1, _S),
                 gnext.reshape(1, _S), bnext.reshape(1, _S)]
        out_specs = bspec(_W)
        out_shape = jax.ShapeDtypeStruct((_N, _W), jnp.float32)
    return pl.pallas_call(
        functools.partial(_node_upd_body, last),
        grid=grid,
        in_specs=in_specs,
        out_specs=out_specs,
        out_shape=out_shape,
    )(*args)


def _build_buckets(dst):
    e_iota = jnp.arange(_E, dtype=jnp.int32)
    bucket = dst // _BKT
    sb, perm = lax.sort_key_val(bucket, e_iota)
    qs5 = jnp.arange(_NB + 1, dtype=jnp.int32)
    off_c = jnp.sum(sb[None, :] < qs5[:, None], axis=1).astype(jnp.int32)
    cnt = off_c[1:] - off_c[:-1]
    cnt_pad = ((cnt + _CPB - 1) // _CPB) * _CPB
    off_pad = jnp.concatenate([jnp.zeros((1,), jnp.int32), jnp.cumsum(cnt_pad).astype(jnp.int32)])
    qs = jnp.arange(_EL, dtype=jnp.int32)
    bq = jnp.sum(qs[:, None] >= off_pad[None, 1:_NB], axis=1).astype(jnp.int32)
    rank = qs - off_pad[bq]
    valid = rank < cnt[bq]
    srci = jnp.clip(off_c[bq] + rank, 0, _E - 1)
    eids = jnp.where(valid, perm[srci], qs % _E)
    dstp = dst[perm]
    ldst = jnp.where(valid, dstp[srci] - _BKT * bq, _BKT + (qs % (_ACC - _BKT)))
    offs = jnp.zeros((16,), jnp.int32).at[:_NB + 1].set(off_pad)
    return eids, ldst, offs


def _forward(P, s, v, d, a, edge_index):
    n = s.shape[0]
    src = edge_index[0]
    dst = edge_index[1]
    eids, ldst, offs = _build_buckets(dst)
    zrow = jnp.zeros((_GCH, _W), jnp.float32)
    padi = (jnp.arange(_EPAD - _E, dtype=jnp.int32) % _N)
    src_p = jnp.concatenate([src, padi])
    dst_p = jnp.concatenate([dst, padi])
    d2 = jnp.concatenate([d, jnp.zeros((_EPAD - _E,), jnp.float32)])[:, None]
    a_p = jnp.concatenate([a, jnp.zeros((_EPAD - _E, 3), jnp.float32)], axis=0)

    deg = jax.ops.segment_sum(jnp.ones((_E,), jnp.float32), dst, num_segments=n)
    rdeg = (1.0 / jnp.maximum(deg, 1.0))[:, None]
    vflat = v.reshape(n, 3 * _V)
    tab = _node_pre(s, vflat, P['gamma0'], P['beta0'])
    for l in range(_L):
        gd, gs = _sc_gather(tab, src_p, dst_p)
        pay = _edge_mlp(l, gd, gs, d2, a_p,
                        P['Wrbf%d' % l], P['W1_%d' % l], P['b1_%d' % l],
                        P['W2_%d' % l], P['b2_%d' % l])
        agg = _sc_scatter(pay, eids, ldst, offs, zrow)
        agg = agg.reshape(_NB, _ACC, _W)[:, :_BKT, :].reshape(n, _W)
        wv3 = jnp.kron(jnp.eye(3, dtype=jnp.float32), P['Wv%d' % l])
        if l < _L - 1:
            tab = _node_upd(tab, agg, rdeg, wv3,
                            P['Wu1_%d' % l], P['bu1_%d' % l],
                            P['Wu2_%d' % l], P['bu2_%d' % l],
                            P['gamma%d' % (l + 1)], P['beta%d' % (l + 1)])
        else:
            s_out, v_out = _node_upd(tab, agg, rdeg, wv3,
                                     None, None, None, None, None, None)
    return s_out, v_out.reshape(n, 3, _V)


def kernel(s, v, edge_index, edge_d, edge_vec, gamma0, beta0, Wrbf0, W1_0, b1_0, W2_0, b2_0, Wv0, Wu1_0, bu1_0, Wu2_0, bu2_0, gamma1, beta1, Wrbf1, W1_1, b1_1, W2_1, b2_1, Wv1, Wu1_1, bu1_1, Wu2_1, bu2_1, gamma2, beta2, Wrbf2, W1_2, b1_2, W2_2, b2_2, Wv2, Wu1_2, bu1_2, Wu2_2, bu2_2, gamma3, beta3, Wrbf3, W1_3, b1_3, W2_3, b2_3, Wv3, Wu1_3, bu1_3, Wu2_3, bu2_3, gamma4, beta4, Wrbf4, W1_4, b1_4, W2_4, b2_4, Wv4):
    kw = dict(locals())
    edge_index = kw.pop('edge_index')
    s = kw.pop('s')
    v = kw.pop('v')
    d = kw.pop('edge_d')
    a = kw.pop('edge_vec')
    return _forward(kw, s, v, d, a, edge_index)


# speedup vs baseline: 20.7672x; 1.0730x over previous
import functools
import jax
import jax.numpy as jnp
import numpy as np
from jax import lax
from jax.experimental import pallas as pl
from jax.experimental.pallas import tpu as pltpu
from jax.experimental.pallas import tpu_sc as plsc

_S = 64
_V = 16
_R = 64
_CUTOFF = 10.0
_L = 5
_N = 50000
_E = 800000

_W = 128
_EB = 2048
_EPAD = 819200
_GW = 32
_GCH = 128
_GNCH = _EPAD // (_GW * _GCH)

_NB = 8
_BKT = 6250
_ACC = 6400
_PT = _ACC // 16
_CPB = 2048
_EL = _E + _NB * _CPB


def _edge_body(l, *refs):
    (gd_ref, gs_ref, d_ref, a0_ref, a1_ref, a2_ref,
     wrbf_ref, w1_ref, b1_ref, w2_ref, b2_ref, out_ref) = refs
    d = d_ref[:, 0:1]
    centers = jax.lax.broadcasted_iota(jnp.int32, (1, _R), 1).astype(jnp.float32) * (_CUTOFF / (_R - 1))
    width = _CUTOFF / _R
    rbf = jnp.exp(-0.5 * ((d - centers) / width) ** 2)
    env = 0.5 * (jnp.cos(jnp.pi * jnp.clip(d, 0.0, _CUTOFF) / _CUTOFF) + 1.0)
    filt = jnp.dot(rbf, wrbf_ref[...], preferred_element_type=jnp.float32)
    sd = gd_ref[:, 0:_S]
    ss = gs_ref[:, 0:_S]
    a0 = a0_ref[...]
    a1 = a1_ref[...]
    a2 = a2_ref[...]
    if l > 0:
        vs = gs_ref[:, _S:_S + 3 * _V]
        vdot = vs[:, 0:16] * a0 + vs[:, 16:32] * a1 + vs[:, 32:48] * a2
        m_in = jnp.concatenate([sd, ss, vdot], axis=1)
    else:
        m_in = jnp.concatenate([sd, ss], axis=1)
    h = m_in @ w1_ref[...] + b1_ref[...]
    h = h * jax.nn.sigmoid(h)
    h = h * filt
    o = h @ w2_ref[...] + b2_ref[...]
    ms = o[:, :_S] * env
    g0 = o[:, _S:_S + _V]
    g1 = o[:, _S + _V:]
    if l > 0:
        vm = jnp.concatenate([g0 * a0 + g1 * vs[:, 0:16],
                              g0 * a1 + g1 * vs[:, 16:32],
                              g0 * a2 + g1 * vs[:, 32:48]], axis=1)
    else:
        vm = jnp.concatenate([g0 * a0, g0 * a1, g0 * a2], axis=1)
    pad = jnp.zeros((_EB, _W - _S - 3 * _V), jnp.float32)
    out_ref[...] = jnp.concatenate([ms, vm * env, pad], axis=1)


def _edge_mlp(l, gd, gs, d2, a0, a1, a2, wrbf, w1, b1, w2, b2):
    grid = (_EPAD // _EB,)
    din = 2 * _S + (_V if l > 0 else 0)
    bspec = lambda w: pl.BlockSpec((_EB, w), lambda i: (i, 0))
    wspec = lambda r, c: pl.BlockSpec((r, c), lambda i: (0, 0))
    in_specs = [bspec(_W), bspec(_W), bspec(1), bspec(1), bspec(1), bspec(1),
                wspec(_R, _S), wspec(din, _S), wspec(1, _S),
                wspec(_S, _S + 2 * _V), wspec(1, _S + 2 * _V)]
    args = [gd, gs, d2, a0, a1, a2, wrbf, w1, b1.reshape(1, _S), w2, b2.reshape(1, _S + 2 * _V)]
    return pl.pallas_call(
        functools.partial(_edge_body, l),
        grid=grid,
        in_specs=in_specs,
        out_specs=pl.BlockSpec((_EB, _W), lambda i: (i, 0)),
        out_shape=jax.ShapeDtypeStruct((_EPAD, _W), jnp.float32),
    )(*args)


def _sc_mesh():
    return plsc.VectorSubcoreMesh(core_axis_name="c", subcore_axis_name="s")


def _gather_body(tab, srci, dsti, gd, gs, idx_s, idx_d, bufd, bufs, sem):
    wid = lax.axis_index("s") * 2 + lax.axis_index("c")
    base = wid * (_GNCH * _GCH)

    def body(j, carry):
        st = base + j * _GCH
        pltpu.sync_copy(srci.at[pl.ds(st, _GCH)], idx_s)
        pltpu.sync_copy(dsti.at[pl.ds(st, _GCH)], idx_d)
        c1 = pltpu.async_copy(tab.at[idx_d], bufd, sem)
        c2 = pltpu.async_copy(tab.at[idx_s], bufs, sem)
        c1.wait()
        c2.wait()
        pltpu.sync_copy(bufd, gd.at[pl.ds(st, _GCH)])
        pltpu.sync_copy(bufs, gs.at[pl.ds(st, _GCH)])
        return carry

    lax.fori_loop(0, _GNCH, body, 0)


def _sc_gather(tab, srci, dsti):
    return pl.kernel(
        _gather_body,
        out_type=[jax.ShapeDtypeStruct((_EPAD, _W), jnp.float32),
                  jax.ShapeDtypeStruct((_EPAD, _W), jnp.float32)],
        mesh=_sc_mesh(),
        compiler_params=pltpu.CompilerParams(use_tc_tiling_on_sc=True),
        scratch_types=[
            pltpu.VMEM((_GCH,), jnp.int32),
            pltpu.VMEM((_GCH,), jnp.int32),
            pltpu.VMEM((_GCH, _W), jnp.float32),
            pltpu.VMEM((_GCH, _W), jnp.float32),
            pltpu.SemaphoreType.DMA,
        ],
    )(tab, srci, dsti)


def _scatter_body(pay, eids, ldst, offs, zrow, out, offv, eidb, ldb, pbuf, zbuf, acc, sem):
    c = lax.axis_index("c")
    s = lax.axis_index("s")
    pltpu.sync_copy(offs, offv)
    pltpu.sync_copy(zrow, zbuf)
    ov = offv[...]
    oly = [ov[i] for i in range(_NB + 1)]

    nfull = _PT // _GCH
    rem = _PT % _GCH
    for bb in range(_NB // 2):
        b = c * (_NB // 2) + bb
        off_b = lax.select(c == 0, oly[bb], oly[_NB // 2 + bb])
        off_b1 = lax.select(c == 0, oly[bb + 1], oly[_NB // 2 + bb + 1])
        nch = (off_b1 - off_b) // _GCH
        nj = (nch + 15 - s) // 16
        for k in range(nfull):
            pltpu.sync_copy(zbuf, acc.at[pl.ds(s * _PT + k * _GCH, _GCH)])
        if rem:
            pltpu.sync_copy(zbuf.at[pl.ds(0, rem)],
                            acc.at[pl.ds(s * _PT + nfull * _GCH, rem)])
        plsc.subcore_barrier()

        def body(j, carry):
            st = pl.multiple_of(off_b + (j * 16 + s) * _GCH, _GCH)
            pltpu.sync_copy(eids.at[pl.ds(st, _GCH)], eidb)
            pltpu.sync_copy(ldst.at[pl.ds(st, _GCH)], ldb.at[0])
            pltpu.async_copy(pay.at[eidb], pbuf, sem).wait()
            pltpu.sync_copy(pbuf, acc.at[ldb.at[0]], add=True)
            return carry

        lax.fori_loop(0, nj, body, 0)
        plsc.subcore_barrier()
        ob = b * _ACC + s * _PT
        for k in range(nfull):
            pltpu.sync_copy(acc.at[pl.ds(s * _PT + k * _GCH, _GCH)], pbuf)
            pltpu.sync_copy(pbuf, out.at[pl.ds(ob + k * _GCH, _GCH)])
        if rem:
            pltpu.sync_copy(acc.at[pl.ds(s * _PT + nfull * _GCH, rem)],
                            pbuf.at[pl.ds(0, rem)])
            pltpu.sync_copy(pbuf.at[pl.ds(0, rem)],
                            out.at[pl.ds(ob + nfull * _GCH, rem)])
        plsc.subcore_barrier()


def _sc_scatter(pay, eids, ldst, offs, zrow):
    return pl.kernel(
        _scatter_body,
        out_type=jax.ShapeDtypeStruct((_NB * _ACC, _W), jnp.float32),
        mesh=_sc_mesh(),
        compiler_params=pltpu.CompilerParams(use_tc_tiling_on_sc=True),
        scratch_types=[
            pltpu.VMEM((16,), jnp.int32),
            pltpu.VMEM((_GCH,), jnp.int32),
            pltpu.VMEM((1, _GCH), jnp.int32),
            pltpu.VMEM((_GCH, _W), jnp.float32),
            pltpu.VMEM((_GCH, _W), jnp.float32),
            pltpu.VMEM_SHARED((_ACC, _W), jnp.float32),
            pltpu.SemaphoreType.DMA,
        ],
    )(pay, eids, ldst, offs, zrow)


_NBLK = 2000


def _norm_tab(s1, v1, g_ref, b_ref):
    mu = jnp.mean(s1, axis=-1, keepdims=True)
    var = jnp.mean((s1 - mu) ** 2, axis=-1, keepdims=True)
    sn = (s1 - mu) / jnp.sqrt(var + 1e-6) * g_ref[...] + b_ref[...]
    vn = jnp.sqrt(jnp.mean(v1 * v1, axis=1, keepdims=True) + 1e-6)
    vnf = v1 / vn
    pad = jnp.zeros((s1.shape[0], _W - _S - 3 * _V), jnp.float32)
    return jnp.concatenate([sn, vnf, pad], axis=1)


def _node_pre_body(s_ref, v_ref, g_ref, b_ref, tab_ref):
    tab_ref[...] = _norm_tab(s_ref[...], v_ref[...], g_ref, b_ref)


def _node_pre(s, vflat, gamma, beta):
    grid = (_N // _NBLK,)
    return pl.pallas_call(
        _node_pre_body,
        grid=grid,
        in_specs=[pl.BlockSpec((_NBLK, _S), lambda i: (i, 0)),
                  pl.BlockSpec((_NBLK, 3 * _V), lambda i: (i, 0)),
                  pl.BlockSpec((1, _S), lambda i: (0, 0)),
                  pl.BlockSpec((1, _S), lambda i: (0, 0))],
        out_specs=pl.BlockSpec((_NBLK, _W), lambda i: (i, 0)),
        out_shape=jax.ShapeDtypeStruct((_N, _W), jnp.float32),
    )(s, vflat, gamma.reshape(1, _S), beta.reshape(1, _S))


def _node_upd_body(last, *refs):
    if last:
        (tab_ref, agg_ref, rdeg_ref, wv3_ref, s_out, v_out) = refs
    else:
        (tab_ref, agg_ref, rdeg_ref, wv3_ref,
         wu1_ref, bu1_ref, wu2_ref, bu2_ref, g_ref, b_ref, tab_out) = refs
    tab = tab_ref[...]
    agg = agg_ref[...]
    rdeg = rdeg_ref[...]
    s1 = tab[:, 0:_S] + agg[:, 0:_S]
    va = agg[:, _S:_S + 3 * _V] * rdeg
    v1 = tab[:, _S:_S + 3 * _V] + jnp.dot(va, wv3_ref[...], preferred_element_type=jnp.float32)
    if last:
        s_out[...] = s1
        v_out[...] = v1
    else:
        h = s1 @ wu1_ref[...] + bu1_ref[...]
        h = h * jax.nn.sigmoid(h)
        s1 = s1 + h @ wu2_ref[...] + bu2_ref[...]
        tab_out[...] = _norm_tab(s1, v1, g_ref, b_ref)


def _node_upd(tab, agg, rdeg, wv3, wu1, bu1, wu2, bu2, gnext, bnext):
    last = wu1 is None
    grid = (_N // _NBLK,)
    bspec = lambda w: pl.BlockSpec((_NBLK, w), lambda i: (i, 0))
    wspec = lambda r, c: pl.BlockSpec((r, c), lambda i: (0, 0))
    in_specs = [bspec(_W), bspec(_W), bspec(1), wspec(3 * _V, 3 * _V)]
    args = [tab, agg, rdeg, wv3]
    if last:
        out_specs = [bspec(_S), bspec(3 * _V)]
        out_shape = [jax.ShapeDtypeStruct((_N, _S), jnp.float32),
                     jax.ShapeDtypeStruct((_N, 3 * _V), jnp.float32)]
    else:
        in_specs += [wspec(_S, _S), wspec(1, _S), wspec(_S, _S), wspec(1, _S),
                     wspec(1, _S), wspec(1, _S)]
        args += [wu1, bu1.reshape(1, _S), wu2, bu2.reshape(1, _S),
                 gnext.reshape(1, _S), bnext.reshape(1, _S)]
        out_specs = bspec(_W)
        out_shape = jax.ShapeDtypeStruct((_N, _W), jnp.float32)
    return pl.pallas_call(
        functools.partial(_node_upd_body, last),
        grid=grid,
        in_specs=in_specs,
        out_specs=out_specs,
        out_shape=out_shape,
    )(*args)


def _build_buckets(dst):
    e_iota = jnp.arange(_E, dtype=jnp.int32)
    bucket = dst // _BKT
    sb, perm = lax.sort_key_val(bucket, e_iota)
    qs5 = jnp.arange(_NB + 1, dtype=jnp.int32)
    off_c = jnp.sum(sb[None, :] < qs5[:, None], axis=1).astype(jnp.int32)
    cnt = off_c[1:] - off_c[:-1]
    cnt_pad = ((cnt + _CPB - 1) // _CPB) * _CPB
    off_pad = jnp.concatenate([jnp.zeros((1,), jnp.int32), jnp.cumsum(cnt_pad).astype(jnp.int32)])
    qs = jnp.arange(_EL, dtype=jnp.int32)
    bq = jnp.sum(qs[:, None] >= off_pad[None, 1:_NB], axis=1).astype(jnp.int32)
    rank = qs - off_pad[bq]
    valid = rank < cnt[bq]
    srci = jnp.clip(off_c[bq] + rank, 0, _E - 1)
    eids = jnp.where(valid, perm[srci], qs % _E)
    dstp = dst[perm]
    ldst = jnp.where(valid, dstp[srci] - _BKT * bq, _BKT + (qs % (_ACC - _BKT)))
    offs = jnp.zeros((16,), jnp.int32).at[:_NB + 1].set(off_pad)
    return eids, ldst, offs


def _forward(P, s, v, d, a, edge_index):
    n = s.shape[0]
    src = edge_index[0]
    dst = edge_index[1]
    eids, ldst, offs = _build_buckets(dst)
    zrow = jnp.zeros((_GCH, _W), jnp.float32)
    padi = (jnp.arange(_EPAD - _E, dtype=jnp.int32) % _N)
    src_p = jnp.concatenate([src, padi])
    dst_p = jnp.concatenate([dst, padi])
    zpad1 = jnp.zeros((_EPAD - _E,), jnp.float32)
    d2 = jnp.concatenate([d, zpad1])[:, None]
    a0 = jnp.concatenate([a[:, 0], zpad1])[:, None]
    a1 = jnp.concatenate([a[:, 1], zpad1])[:, None]
    a2 = jnp.concatenate([a[:, 2], zpad1])[:, None]

    deg = jax.ops.segment_sum(jnp.ones((_E,), jnp.float32), dst, num_segments=n)
    rdeg = (1.0 / jnp.maximum(deg, 1.0))[:, None]
    vflat = v.reshape(n, 3 * _V)
    tab = _node_pre(s, vflat, P['gamma0'], P['beta0'])
    for l in range(_L):
        gd, gs = _sc_gather(tab, src_p, dst_p)
        pay = _edge_mlp(l, gd, gs, d2, a0, a1, a2,
                        P['Wrbf%d' % l], P['W1_%d' % l], P['b1_%d' % l],
                        P['W2_%d' % l], P['b2_%d' % l])
        agg = _sc_scatter(pay, eids, ldst, offs, zrow)
        agg = agg.reshape(_NB, _ACC, _W)[:, :_BKT, :].reshape(n, _W)
        wv3 = jnp.kron(jnp.eye(3, dtype=jnp.float32), P['Wv%d' % l])
        if l < _L - 1:
            tab = _node_upd(tab, agg, rdeg, wv3,
                            P['Wu1_%d' % l], P['bu1_%d' % l],
                            P['Wu2_%d' % l], P['bu2_%d' % l],
                            P['gamma%d' % (l + 1)], P['beta%d' % (l + 1)])
        else:
            s_out, v_out = _node_upd(tab, agg, rdeg, wv3,
                                     None, None, None, None, None, None)
    return s_out, v_out.reshape(n, 3, _V)


def kernel(s, v, edge_index, edge_d, edge_vec, gamma0, beta0, Wrbf0, W1_0, b1_0, W2_0, b2_0, Wv0, Wu1_0, bu1_0, Wu2_0, bu2_0, gamma1, beta1, Wrbf1, W1_1, b1_1, W2_1, b2_1, Wv1, Wu1_1, bu1_1, Wu2_1, bu2_1, gamma2, beta2, Wrbf2, W1_2, b1_2, W2_2, b2_2, Wv2, Wu1_2, bu1_2, Wu2_2, bu2_2, gamma3, beta3, Wrbf3, W1_3, b1_3, W2_3, b2_3, Wv3, Wu1_3, bu1_3, Wu2_3, bu2_3, gamma4, beta4, Wrbf4, W1_4, b1_4, W2_4, b2_4, Wv4):
    kw = dict(locals())
    edge_index = kw.pop('edge_index')
    s = kw.pop('s')
    v = kw.pop('v')
    d = kw.pop('edge_d')
    a = kw.pop('edge_vec')
    return _forward(kw, s, v, d, a, edge_index)


# speedup vs baseline: 21.8319x; 1.0513x over previous
import functools
import jax
import jax.numpy as jnp
import numpy as np
from jax import lax
from jax.experimental import pallas as pl
from jax.experimental.pallas import tpu as pltpu
from jax.experimental.pallas import tpu_sc as plsc

_S = 64
_V = 16
_R = 64
_CUTOFF = 10.0
_L = 5
_N = 50000
_E = 800000

_W = 128
_EB = 2048
_EPAD = 819200
_GW = 32
_GCH = 128
_GNCH = _EPAD // (_GW * _GCH)

_NB = 8
_BKT = 6250
_ACC = 6400
_PT = _ACC // 16
_CPB = 2048
_EL = _E + _NB * _CPB


def _edge_body(l, *refs):
    (gd_ref, gs_ref, d_ref, a0_ref, a1_ref, a2_ref,
     wrbf_ref, w1_ref, b1_ref, w2_ref, b2_ref, out_ref) = refs
    d = d_ref[:, 0:1]
    centers = jax.lax.broadcasted_iota(jnp.int32, (1, _R), 1).astype(jnp.float32) * (_CUTOFF / (_R - 1))
    width = _CUTOFF / _R
    rbf = jnp.exp(-0.5 * ((d - centers) / width) ** 2)
    env = 0.5 * (jnp.cos(jnp.pi * jnp.clip(d, 0.0, _CUTOFF) / _CUTOFF) + 1.0)
    filt = jnp.dot(rbf, wrbf_ref[...], preferred_element_type=jnp.float32)
    sd = gd_ref[:, 0:_S]
    ss = gs_ref[:, 0:_S]
    a0 = a0_ref[...]
    a1 = a1_ref[...]
    a2 = a2_ref[...]
    if l > 0:
        vs = gs_ref[:, _S:_S + 3 * _V]
        vdot = vs[:, 0:16] * a0 + vs[:, 16:32] * a1 + vs[:, 32:48] * a2
        m_in = jnp.concatenate([sd, ss, vdot], axis=1)
    else:
        m_in = jnp.concatenate([sd, ss], axis=1)
    h = m_in @ w1_ref[...] + b1_ref[...]
    h = h * jax.nn.sigmoid(h)
    h = h * filt
    o = h @ w2_ref[...] + b2_ref[...]
    ms = o[:, :_S] * env
    g0 = o[:, _S:_S + _V]
    g1 = o[:, _S + _V:]
    if l > 0:
        vm = jnp.concatenate([g0 * a0 + g1 * vs[:, 0:16],
                              g0 * a1 + g1 * vs[:, 16:32],
                              g0 * a2 + g1 * vs[:, 32:48]], axis=1)
    else:
        vm = jnp.concatenate([g0 * a0, g0 * a1, g0 * a2], axis=1)
    pad = jnp.zeros((_EB, _W - _S - 3 * _V), jnp.float32)
    out_ref[...] = jnp.concatenate([ms, vm * env, pad], axis=1)


def _edge_mlp(l, gd, gs, d2, a0, a1, a2, wrbf, w1, b1, w2, b2):
    grid = (_EPAD // _EB,)
    din = 2 * _S + (_V if l > 0 else 0)
    bspec = lambda w: pl.BlockSpec((_EB, w), lambda i: (i, 0))
    wspec = lambda r, c: pl.BlockSpec((r, c), lambda i: (0, 0))
    in_specs = [bspec(_W), bspec(_W), bspec(1), bspec(1), bspec(1), bspec(1),
                wspec(_R, _S), wspec(din, _S), wspec(1, _S),
                wspec(_S, _S + 2 * _V), wspec(1, _S + 2 * _V)]
    args = [gd, gs, d2, a0, a1, a2, wrbf, w1, b1.reshape(1, _S), w2, b2.reshape(1, _S + 2 * _V)]
    return pl.pallas_call(
        functools.partial(_edge_body, l),
        grid=grid,
        in_specs=in_specs,
        out_specs=pl.BlockSpec((_EB, _W), lambda i: (i, 0)),
        out_shape=jax.ShapeDtypeStruct((_EPAD, _W), jnp.float32),
    )(*args)


def _sc_mesh():
    return plsc.VectorSubcoreMesh(core_axis_name="c", subcore_axis_name="s")


def _gather_body(tab, srci, dsti, gd, gs,
                 ix0s, ix0d, ix1s, ix1d, bd0, bs0, bd1, bs1, sg0, sg1, so0, so1):
    wid = lax.axis_index("s") * 2 + lax.axis_index("c")
    base = wid * (_GNCH * _GCH)
    IX = ((ix0s, ix0d), (ix1s, ix1d))
    BF = ((bd0, bs0), (bd1, bs1))
    SG = (sg0, sg1)
    SO = (so0, so1)

    def load_idx(j, b):
        st = base + j * _GCH
        pltpu.sync_copy(srci.at[pl.ds(st, _GCH)], IX[b][0])
        pltpu.sync_copy(dsti.at[pl.ds(st, _GCH)], IX[b][1])

    def fire_g(b):
        pltpu.async_copy(tab.at[IX[b][1]], BF[b][0], SG[b])
        pltpu.async_copy(tab.at[IX[b][0]], BF[b][1], SG[b])

    def wait_g(b):
        pltpu.make_async_copy(tab.at[IX[b][1]], BF[b][0], SG[b]).wait()
        pltpu.make_async_copy(tab.at[IX[b][0]], BF[b][1], SG[b]).wait()

    def fire_o(j, b):
        st = base + j * _GCH
        pltpu.async_copy(BF[b][0], gd.at[pl.ds(st, _GCH)], SO[b])
        pltpu.async_copy(BF[b][1], gs.at[pl.ds(st, _GCH)], SO[b])

    def wait_o(j, b):
        st = base + j * _GCH
        pltpu.make_async_copy(BF[b][0], gd.at[pl.ds(st, _GCH)], SO[b]).wait()
        pltpu.make_async_copy(BF[b][1], gs.at[pl.ds(st, _GCH)], SO[b]).wait()

    load_idx(0, 0)
    fire_g(0)
    load_idx(1, 1)
    fire_g(1)

    def body(i, carry):
        for b in range(2):
            j = 2 * i + b
            wait_g(b)
            fire_o(j, b)
        for b in range(2):
            jn = 2 * i + 2 + b
            wait_o(jn - 2, b)
            load_idx(jn, b)
            fire_g(b)
        return carry

    lax.fori_loop(0, _GNCH // 2 - 1, body, 0)
    for b in range(2):
        wait_g(b)
        fire_o(_GNCH - 2 + b, b)
    for b in range(2):
        wait_o(_GNCH - 2 + b, b)


def _sc_gather(tab, srci, dsti):
    return pl.kernel(
        _gather_body,
        out_type=[jax.ShapeDtypeStruct((_EPAD, _W), jnp.float32),
                  jax.ShapeDtypeStruct((_EPAD, _W), jnp.float32)],
        mesh=_sc_mesh(),
        compiler_params=pltpu.CompilerParams(use_tc_tiling_on_sc=True),
        scratch_types=[
            pltpu.VMEM((_GCH,), jnp.int32),
            pltpu.VMEM((_GCH,), jnp.int32),
            pltpu.VMEM((_GCH,), jnp.int32),
            pltpu.VMEM((_GCH,), jnp.int32),
            pltpu.VMEM((_GCH, _W), jnp.float32),
            pltpu.VMEM((_GCH, _W), jnp.float32),
            pltpu.VMEM((_GCH, _W), jnp.float32),
            pltpu.VMEM((_GCH, _W), jnp.float32),
            pltpu.SemaphoreType.DMA,
            pltpu.SemaphoreType.DMA,
            pltpu.SemaphoreType.DMA,
            pltpu.SemaphoreType.DMA,
        ],
    )(tab, srci, dsti)


def _scatter_body(pay, eids, ldst, offs, zrow, out, offv, eidb, ldb, pbuf, zbuf, acc, sem):
    c = lax.axis_index("c")
    s = lax.axis_index("s")
    pltpu.sync_copy(offs, offv)
    pltpu.sync_copy(zrow, zbuf)
    ov = offv[...]
    oly = [ov[i] for i in range(_NB + 1)]

    nfull = _PT // _GCH
    rem = _PT % _GCH
    for bb in range(_NB // 2):
        b = c * (_NB // 2) + bb
        off_b = lax.select(c == 0, oly[bb], oly[_NB // 2 + bb])
        off_b1 = lax.select(c == 0, oly[bb + 1], oly[_NB // 2 + bb + 1])
        nch = (off_b1 - off_b) // _GCH
        nj = (nch + 15 - s) // 16
        for k in range(nfull):
            pltpu.sync_copy(zbuf, acc.at[pl.ds(s * _PT + k * _GCH, _GCH)])
        if rem:
            pltpu.sync_copy(zbuf.at[pl.ds(0, rem)],
                            acc.at[pl.ds(s * _PT + nfull * _GCH, rem)])
        plsc.subcore_barrier()

        def body(j, carry):
            st = pl.multiple_of(off_b + (j * 16 + s) * _GCH, _GCH)
            pltpu.sync_copy(eids.at[pl.ds(st, _GCH)], eidb)
            pltpu.sync_copy(ldst.at[pl.ds(st, _GCH)], ldb.at[0])
            pltpu.async_copy(pay.at[eidb], pbuf, sem).wait()
            pltpu.sync_copy(pbuf, acc.at[ldb.at[0]], add=True)
            return carry

        lax.fori_loop(0, nj, body, 0)
        plsc.subcore_barrier()
        ob = b * _ACC + s * _PT
        for k in range(nfull):
            pltpu.sync_copy(acc.at[pl.ds(s * _PT + k * _GCH, _GCH)], pbuf)
            pltpu.sync_copy(pbuf, out.at[pl.ds(ob + k * _GCH, _GCH)])
        if rem:
            pltpu.sync_copy(acc.at[pl.ds(s * _PT + nfull * _GCH, rem)],
                            pbuf.at[pl.ds(0, rem)])
            pltpu.sync_copy(pbuf.at[pl.ds(0, rem)],
                            out.at[pl.ds(ob + nfull * _GCH, rem)])
        plsc.subcore_barrier()


def _sc_scatter(pay, eids, ldst, offs, zrow):
    return pl.kernel(
        _scatter_body,
        out_type=jax.ShapeDtypeStruct((_NB * _ACC, _W), jnp.float32),
        mesh=_sc_mesh(),
        compiler_params=pltpu.CompilerParams(use_tc_tiling_on_sc=True),
        scratch_types=[
            pltpu.VMEM((16,), jnp.int32),
            pltpu.VMEM((_GCH,), jnp.int32),
            pltpu.VMEM((1, _GCH), jnp.int32),
            pltpu.VMEM((_GCH, _W), jnp.float32),
            pltpu.VMEM((_GCH, _W), jnp.float32),
            pltpu.VMEM_SHARED((_ACC, _W), jnp.float32),
            pltpu.SemaphoreType.DMA,
        ],
    )(pay, eids, ldst, offs, zrow)


_NBLK = 2000


def _norm_tab(s1, v1, g_ref, b_ref):
    mu = jnp.mean(s1, axis=-1, keepdims=True)
    var = jnp.mean((s1 - mu) ** 2, axis=-1, keepdims=True)
    sn = (s1 - mu) / jnp.sqrt(var + 1e-6) * g_ref[...] + b_ref[...]
    vn = jnp.sqrt(jnp.mean(v1 * v1, axis=1, keepdims=True) + 1e-6)
    vnf = v1 / vn
    pad = jnp.zeros((s1.shape[0], _W - _S - 3 * _V), jnp.float32)
    return jnp.concatenate([sn, vnf, pad], axis=1)


def _node_pre_body(s_ref, v_ref, g_ref, b_ref, tab_ref):
    tab_ref[...] = _norm_tab(s_ref[...], v_ref[...], g_ref, b_ref)


def _node_pre(s, vflat, gamma, beta):
    grid = (_N // _NBLK,)
    return pl.pallas_call(
        _node_pre_body,
        grid=grid,
        in_specs=[pl.BlockSpec((_NBLK, _S), lambda i: (i, 0)),
                  pl.BlockSpec((_NBLK, 3 * _V), lambda i: (i, 0)),
                  pl.BlockSpec((1, _S), lambda i: (0, 0)),
                  pl.BlockSpec((1, _S), lambda i: (0, 0))],
        out_specs=pl.BlockSpec((_NBLK, _W), lambda i: (i, 0)),
        out_shape=jax.ShapeDtypeStruct((_N, _W), jnp.float32),
    )(s, vflat, gamma.reshape(1, _S), beta.reshape(1, _S))


def _node_upd_body(last, *refs):
    if last:
        (tab_ref, agg_ref, rdeg_ref, wv3_ref, s_out, v_out) = refs
    else:
        (tab_ref, agg_ref, rdeg_ref, wv3_ref,
         wu1_ref, bu1_ref, wu2_ref, bu2_ref, g_ref, b_ref, tab_out) = refs
    tab = tab_ref[...]
    agg = agg_ref[...]
    rdeg = rdeg_ref[...]
    s1 = tab[:, 0:_S] + agg[:, 0:_S]
    va = agg[:, _S:_S + 3 * _V] * rdeg
    v1 = tab[:, _S:_S + 3 * _V] + jnp.dot(va, wv3_ref[...], preferred_element_type=jnp.float32)
    if last:
        s_out[...] = s1
        v_out[...] = v1
    else:
        h = s1 @ wu1_ref[...] + bu1_ref[...]
        h = h * jax.nn.sigmoid(h)
        s1 = s1 + h @ wu2_ref[...] + bu2_ref[...]
        tab_out[...] = _norm_tab(s1, v1, g_ref, b_ref)


def _node_upd(tab, agg, rdeg, wv3, wu1, bu1, wu2, bu2, gnext, bnext):
    last = wu1 is None
    grid = (_N // _NBLK,)
    bspec = lambda w: pl.BlockSpec((_NBLK, w), lambda i: (i, 0))
    wspec = lambda r, c: pl.BlockSpec((r, c), lambda i: (0, 0))
    in_specs = [bspec(_W), bspec(_W), bspec(1), wspec(3 * _V, 3 * _V)]
    args = [tab, agg, rdeg, wv3]
    if last:
        out_specs = [bspec(_S), bspec(3 * _V)]
        out_shape = [jax.ShapeDtypeStruct((_N, _S), jnp.float32),
                     jax.ShapeDtypeStruct((_N, 3 * _V), jnp.float32)]
    else:
        in_specs += [wspec(_S, _S), wspec(1, _S), wspec(_S, _S), wspec(1, _S),
                     wspec(1, _S), wspec(1, _S)]
        args += [wu1, bu1.reshape(1, _S), wu2, bu2.reshape(1, _S),
                 gnext.reshape(1, _S), bnext.reshape(1, _S)]
        out_specs = bspec(_W)
        out_shape = jax.ShapeDtypeStruct((_N, _W), jnp.float32)
    return pl.pallas_call(
        functools.partial(_node_upd_body, last),
        grid=grid,
        in_specs=in_specs,
        out_specs=out_specs,
        out_shape=out_shape,
    )(*args)


def _build_buckets(dst):
    e_iota = jnp.arange(_E, dtype=jnp.int32)
    bucket = dst // _BKT
    sb, perm = lax.sort_key_val(bucket, e_iota)
    qs5 = jnp.arange(_NB + 1, dtype=jnp.int32)
    off_c = jnp.sum(sb[None, :] < qs5[:, None], axis=1).astype(jnp.int32)
    cnt = off_c[1:] - off_c[:-1]
    cnt_pad = ((cnt + _CPB - 1) // _CPB) * _CPB
    off_pad = jnp.concatenate([jnp.zeros((1,), jnp.int32), jnp.cumsum(cnt_pad).astype(jnp.int32)])
    qs = jnp.arange(_EL, dtype=jnp.int32)
    bq = jnp.sum(qs[:, None] >= off_pad[None, 1:_NB], axis=1).astype(jnp.int32)
    rank = qs - off_pad[bq]
    valid = rank < cnt[bq]
    srci = jnp.clip(off_c[bq] + rank, 0, _E - 1)
    eids = jnp.where(valid, perm[srci], qs % _E)
    dstp = dst[perm]
    ldst = jnp.where(valid, dstp[srci] - _BKT * bq, _BKT + (qs % (_ACC - _BKT)))
    offs = jnp.zeros((16,), jnp.int32).at[:_NB + 1].set(off_pad)
    return eids, ldst, offs


def _forward(P, s, v, d, a, edge_index):
    n = s.shape[0]
    src = edge_index[0]
    dst = edge_index[1]
    eids, ldst, offs = _build_buckets(dst)
    zrow = jnp.zeros((_GCH, _W), jnp.float32)
    padi = (jnp.arange(_EPAD - _E, dtype=jnp.int32) % _N)
    src_p = jnp.concatenate([src, padi])
    dst_p = jnp.concatenate([dst, padi])
    zpad1 = jnp.zeros((_EPAD - _E,), jnp.float32)
    d2 = jnp.concatenate([d, zpad1])[:, None]
    a0 = jnp.concatenate([a[:, 0], zpad1])[:, None]
    a1 = jnp.concatenate([a[:, 1], zpad1])[:, None]
    a2 = jnp.concatenate([a[:, 2], zpad1])[:, None]

    deg = jax.ops.segment_sum(jnp.ones((_E,), jnp.float32), dst, num_segments=n)
    rdeg = (1.0 / jnp.maximum(deg, 1.0))[:, None]
    vflat = v.reshape(n, 3 * _V)
    tab = _node_pre(s, vflat, P['gamma0'], P['beta0'])
    for l in range(_L):
        gd, gs = _sc_gather(tab, src_p, dst_p)
        pay = _edge_mlp(l, gd, gs, d2, a0, a1, a2,
                        P['Wrbf%d' % l], P['W1_%d' % l], P['b1_%d' % l],
                        P['W2_%d' % l], P['b2_%d' % l])
        agg = _sc_scatter(pay, eids, ldst, offs, zrow)
        agg = agg.reshape(_NB, _ACC, _W)[:, :_BKT, :].reshape(n, _W)
        wv3 = jnp.kron(jnp.eye(3, dtype=jnp.float32), P['Wv%d' % l])
        if l < _L - 1:
            tab = _node_upd(tab, agg, rdeg, wv3,
                            P['Wu1_%d' % l], P['bu1_%d' % l],
                            P['Wu2_%d' % l], P['bu2_%d' % l],
                            P['gamma%d' % (l + 1)], P['beta%d' % (l + 1)])
        else:
            s_out, v_out = _node_upd(tab, agg, rdeg, wv3,
                                     None, None, None, None, None, None)
    return s_out, v_out.reshape(n, 3, _V)


def kernel(s, v, edge_index, edge_d, edge_vec, gamma0, beta0, Wrbf0, W1_0, b1_0, W2_0, b2_0, Wv0, Wu1_0, bu1_0, Wu2_0, bu2_0, gamma1, beta1, Wrbf1, W1_1, b1_1, W2_1, b2_1, Wv1, Wu1_1, bu1_1, Wu2_1, bu2_1, gamma2, beta2, Wrbf2, W1_2, b1_2, W2_2, b2_2, Wv2, Wu1_2, bu1_2, Wu2_2, bu2_2, gamma3, beta3, Wrbf3, W1_3, b1_3, W2_3, b2_3, Wv3, Wu1_3, bu1_3, Wu2_3, bu2_3, gamma4, beta4, Wrbf4, W1_4, b1_4, W2_4, b2_4, Wv4):
    kw = dict(locals())
    edge_index = kw.pop('edge_index')
    s = kw.pop('s')
    v = kw.pop('v')
    d = kw.pop('edge_d')
    a = kw.pop('edge_vec')
    return _forward(kw, s, v, d, a, edge_index)


# speedup vs baseline: 22.6895x; 1.0393x over previous
import functools
import jax
import jax.numpy as jnp
import numpy as np
from jax import lax
from jax.experimental import pallas as pl
from jax.experimental.pallas import tpu as pltpu
from jax.experimental.pallas import tpu_sc as plsc

_S = 64
_V = 16
_R = 64
_CUTOFF = 10.0
_L = 5
_N = 50000
_E = 800000

_W = 128
_EB = 2048
_EPAD = 819200
_GW = 32
_GCH = 128
_GNCH = _EPAD // (_GW * _GCH)

_NB = 8
_BKT = 6250
_ACC = 6400
_PT = _ACC // 16
_CPB = 2048
_EL = _E + _NB * _CPB


def _edge_body(l, *refs):
    (gd_ref, gs_ref, d_ref, a0_ref, a1_ref, a2_ref,
     wrbf_ref, w1_ref, b1_ref, w2_ref, b2_ref, out_ref) = refs
    d = d_ref[:, 0:1]
    centers = jax.lax.broadcasted_iota(jnp.int32, (1, _R), 1).astype(jnp.float32) * (_CUTOFF / (_R - 1))
    width = _CUTOFF / _R
    rbf = jnp.exp(-0.5 * ((d - centers) / width) ** 2)
    env = 0.5 * (jnp.cos(jnp.pi * jnp.clip(d, 0.0, _CUTOFF) / _CUTOFF) + 1.0)
    filt = jnp.dot(rbf, wrbf_ref[...], preferred_element_type=jnp.float32)
    sd = gd_ref[:, 0:_S]
    ss = gs_ref[:, 0:_S]
    a0 = a0_ref[...]
    a1 = a1_ref[...]
    a2 = a2_ref[...]
    if l > 0:
        vs = gs_ref[:, _S:_S + 3 * _V]
        vdot = vs[:, 0:16] * a0 + vs[:, 16:32] * a1 + vs[:, 32:48] * a2
        m_in = jnp.concatenate([sd, ss, vdot], axis=1)
    else:
        m_in = jnp.concatenate([sd, ss], axis=1)
    h = m_in @ w1_ref[...] + b1_ref[...]
    h = h * jax.nn.sigmoid(h)
    h = h * filt
    o = h @ w2_ref[...] + b2_ref[...]
    ms = o[:, :_S] * env
    g0 = o[:, _S:_S + _V]
    g1 = o[:, _S + _V:]
    if l > 0:
        vm = jnp.concatenate([g0 * a0 + g1 * vs[:, 0:16],
                              g0 * a1 + g1 * vs[:, 16:32],
                              g0 * a2 + g1 * vs[:, 32:48]], axis=1)
    else:
        vm = jnp.concatenate([g0 * a0, g0 * a1, g0 * a2], axis=1)
    ones = jnp.ones((_EB, 1), jnp.float32)
    pad = jnp.zeros((_EB, _W - _S - 3 * _V - 1), jnp.float32)
    out_ref[...] = jnp.concatenate([ms, vm * env, ones, pad], axis=1)


def _edge_mlp(l, gd, gs, d2, a0, a1, a2, wrbf, w1, b1, w2, b2):
    grid = (_EPAD // _EB,)
    din = 2 * _S + (_V if l > 0 else 0)
    bspec = lambda w: pl.BlockSpec((_EB, w), lambda i: (i, 0))
    wspec = lambda r, c: pl.BlockSpec((r, c), lambda i: (0, 0))
    in_specs = [bspec(_W), bspec(_W), bspec(1), bspec(1), bspec(1), bspec(1),
                wspec(_R, _S), wspec(din, _S), wspec(1, _S),
                wspec(_S, _S + 2 * _V), wspec(1, _S + 2 * _V)]
    args = [gd, gs, d2, a0, a1, a2, wrbf, w1, b1.reshape(1, _S), w2, b2.reshape(1, _S + 2 * _V)]
    return pl.pallas_call(
        functools.partial(_edge_body, l),
        grid=grid,
        in_specs=in_specs,
        out_specs=pl.BlockSpec((_EB, _W), lambda i: (i, 0)),
        out_shape=jax.ShapeDtypeStruct((_EPAD, _W), jnp.float32),
    )(*args)


def _sc_mesh():
    return plsc.VectorSubcoreMesh(core_axis_name="c", subcore_axis_name="s")


def _gather_body(tab, srci, dsti, gd, gs,
                 ix0s, ix0d, ix1s, ix1d, bd0, bs0, bd1, bs1, sg0, sg1, so0, so1):
    wid = lax.axis_index("s") * 2 + lax.axis_index("c")
    base = wid * (_GNCH * _GCH)
    IX = ((ix0s, ix0d), (ix1s, ix1d))
    BF = ((bd0, bs0), (bd1, bs1))
    SG = (sg0, sg1)
    SO = (so0, so1)

    def load_idx(j, b):
        st = base + j * _GCH
        pltpu.sync_copy(srci.at[pl.ds(st, _GCH)], IX[b][0])
        pltpu.sync_copy(dsti.at[pl.ds(st, _GCH)], IX[b][1])

    def fire_g(b):
        pltpu.async_copy(tab.at[IX[b][1]], BF[b][0], SG[b])
        pltpu.async_copy(tab.at[IX[b][0]], BF[b][1], SG[b])

    def wait_g(b):
        pltpu.make_async_copy(tab.at[IX[b][1]], BF[b][0], SG[b]).wait()
        pltpu.make_async_copy(tab.at[IX[b][0]], BF[b][1], SG[b]).wait()

    def fire_o(j, b):
        st = base + j * _GCH
        pltpu.async_copy(BF[b][0], gd.at[pl.ds(st, _GCH)], SO[b])
        pltpu.async_copy(BF[b][1], gs.at[pl.ds(st, _GCH)], SO[b])

    def wait_o(j, b):
        st = base + j * _GCH
        pltpu.make_async_copy(BF[b][0], gd.at[pl.ds(st, _GCH)], SO[b]).wait()
        pltpu.make_async_copy(BF[b][1], gs.at[pl.ds(st, _GCH)], SO[b]).wait()

    load_idx(0, 0)
    fire_g(0)
    load_idx(1, 1)
    fire_g(1)

    def body(i, carry):
        for b in range(2):
            j = 2 * i + b
            wait_g(b)
            fire_o(j, b)
        for b in range(2):
            jn = 2 * i + 2 + b
            wait_o(jn - 2, b)
            load_idx(jn, b)
            fire_g(b)
        return carry

    lax.fori_loop(0, _GNCH // 2 - 1, body, 0)
    for b in range(2):
        wait_g(b)
        fire_o(_GNCH - 2 + b, b)
    for b in range(2):
        wait_o(_GNCH - 2 + b, b)


def _sc_gather(tab, srci, dsti):
    return pl.kernel(
        _gather_body,
        out_type=[jax.ShapeDtypeStruct((_EPAD, _W), jnp.float32),
                  jax.ShapeDtypeStruct((_EPAD, _W), jnp.float32)],
        mesh=_sc_mesh(),
        compiler_params=pltpu.CompilerParams(use_tc_tiling_on_sc=True),
        scratch_types=[
            pltpu.VMEM((_GCH,), jnp.int32),
            pltpu.VMEM((_GCH,), jnp.int32),
            pltpu.VMEM((_GCH,), jnp.int32),
            pltpu.VMEM((_GCH,), jnp.int32),
            pltpu.VMEM((_GCH, _W), jnp.float32),
            pltpu.VMEM((_GCH, _W), jnp.float32),
            pltpu.VMEM((_GCH, _W), jnp.float32),
            pltpu.VMEM((_GCH, _W), jnp.float32),
            pltpu.SemaphoreType.DMA,
            pltpu.SemaphoreType.DMA,
            pltpu.SemaphoreType.DMA,
            pltpu.SemaphoreType.DMA,
        ],
    )(tab, srci, dsti)


def _scatter_body(pay, eids, ldst, offs, zrow, out, offv, eidb, ldb, pbuf, zbuf, acc, sem):
    c = lax.axis_index("c")
    s = lax.axis_index("s")
    pltpu.sync_copy(offs, offv)
    pltpu.sync_copy(zrow, zbuf)
    ov = offv[...]
    oly = [ov[i] for i in range(_NB + 1)]

    nfull = _PT // _GCH
    rem = _PT % _GCH
    for bb in range(_NB // 2):
        b = c * (_NB // 2) + bb
        off_b = lax.select(c == 0, oly[bb], oly[_NB // 2 + bb])
        off_b1 = lax.select(c == 0, oly[bb + 1], oly[_NB // 2 + bb + 1])
        nch = (off_b1 - off_b) // _GCH
        nj = (nch + 15 - s) // 16
        for k in range(nfull):
            pltpu.sync_copy(zbuf, acc.at[pl.ds(s * _PT + k * _GCH, _GCH)])
        if rem:
            pltpu.sync_copy(zbuf.at[pl.ds(0, rem)],
                            acc.at[pl.ds(s * _PT + nfull * _GCH, rem)])
        plsc.subcore_barrier()

        def body(j, carry):
            st = pl.multiple_of(off_b + (j * 16 + s) * _GCH, _GCH)
            pltpu.sync_copy(eids.at[pl.ds(st, _GCH)], eidb)
            pltpu.sync_copy(ldst.at[pl.ds(st, _GCH)], ldb.at[0])
            pltpu.async_copy(pay.at[eidb], pbuf, sem).wait()
            pltpu.sync_copy(pbuf, acc.at[ldb.at[0]], add=True)
            return carry

        lax.fori_loop(0, nj, body, 0)
        plsc.subcore_barrier()
        ob = b * _ACC + s * _PT
        for k in range(nfull):
            pltpu.sync_copy(acc.at[pl.ds(s * _PT + k * _GCH, _GCH)], pbuf)
            pltpu.sync_copy(pbuf, out.at[pl.ds(ob + k * _GCH, _GCH)])
        if rem:
            pltpu.sync_copy(acc.at[pl.ds(s * _PT + nfull * _GCH, rem)],
                            pbuf.at[pl.ds(0, rem)])
            pltpu.sync_copy(pbuf.at[pl.ds(0, rem)],
                            out.at[pl.ds(ob + nfull * _GCH, rem)])
        plsc.subcore_barrier()


def _sc_scatter(pay, eids, ldst, offs, zrow):
    return pl.kernel(
        _scatter_body,
        out_type=jax.ShapeDtypeStruct((_NB * _ACC, _W), jnp.float32),
        mesh=_sc_mesh(),
        compiler_params=pltpu.CompilerParams(use_tc_tiling_on_sc=True),
        scratch_types=[
            pltpu.VMEM((16,), jnp.int32),
            pltpu.VMEM((_GCH,), jnp.int32),
            pltpu.VMEM((1, _GCH), jnp.int32),
            pltpu.VMEM((_GCH, _W), jnp.float32),
            pltpu.VMEM((_GCH, _W), jnp.float32),
            pltpu.VMEM_SHARED((_ACC, _W), jnp.float32),
            pltpu.SemaphoreType.DMA,
        ],
    )(pay, eids, ldst, offs, zrow)


_NBLK = 2000


def _norm_tab(s1, v1, g_ref, b_ref):
    mu = jnp.mean(s1, axis=-1, keepdims=True)
    var = jnp.mean((s1 - mu) ** 2, axis=-1, keepdims=True)
    sn = (s1 - mu) / jnp.sqrt(var + 1e-6) * g_ref[...] + b_ref[...]
    vn = jnp.sqrt(jnp.mean(v1 * v1, axis=1, keepdims=True) + 1e-6)
    vnf = v1 / vn
    pad = jnp.zeros((s1.shape[0], _W - _S - 3 * _V), jnp.float32)
    return jnp.concatenate([sn, vnf, pad], axis=1)


def _node_pre_body(s_ref, v_ref, g_ref, b_ref, tab_ref):
    tab_ref[...] = _norm_tab(s_ref[...], v_ref[...], g_ref, b_ref)


def _node_pre(s, vflat, gamma, beta):
    grid = (_N // _NBLK,)
    return pl.pallas_call(
        _node_pre_body,
        grid=grid,
        in_specs=[pl.BlockSpec((_NBLK, _S), lambda i: (i, 0)),
                  pl.BlockSpec((_NBLK, 3 * _V), lambda i: (i, 0)),
                  pl.BlockSpec((1, _S), lambda i: (0, 0)),
                  pl.BlockSpec((1, _S), lambda i: (0, 0))],
        out_specs=pl.BlockSpec((_NBLK, _W), lambda i: (i, 0)),
        out_shape=jax.ShapeDtypeStruct((_N, _W), jnp.float32),
    )(s, vflat, gamma.reshape(1, _S), beta.reshape(1, _S))


def _node_upd_body(last, *refs):
    if last:
        (tab_ref, agg_ref, rdeg_ref, wv3_ref, s_out, v_out) = refs
    else:
        (tab_ref, agg_ref, rdeg_ref, wv3_ref,
         wu1_ref, bu1_ref, wu2_ref, bu2_ref, g_ref, b_ref, tab_out) = refs
    tab = tab_ref[...]
    agg = agg_ref[...]
    rdeg = rdeg_ref[...]
    s1 = tab[:, 0:_S] + agg[:, 0:_S]
    va = agg[:, _S:_S + 3 * _V] * rdeg
    v1 = tab[:, _S:_S + 3 * _V] + jnp.dot(va, wv3_ref[...], preferred_element_type=jnp.float32)
    if last:
        s_out[...] = s1
        v_out[...] = v1
    else:
        h = s1 @ wu1_ref[...] + bu1_ref[...]
        h = h * jax.nn.sigmoid(h)
        s1 = s1 + h @ wu2_ref[...] + bu2_ref[...]
        tab_out[...] = _norm_tab(s1, v1, g_ref, b_ref)


def _node_upd(tab, agg, rdeg, wv3, wu1, bu1, wu2, bu2, gnext, bnext):
    last = wu1 is None
    grid = (_N // _NBLK,)
    bspec = lambda w: pl.BlockSpec((_NBLK, w), lambda i: (i, 0))
    wspec = lambda r, c: pl.BlockSpec((r, c), lambda i: (0, 0))
    in_specs = [bspec(_W), bspec(_W), bspec(1), wspec(3 * _V, 3 * _V)]
    args = [tab, agg, rdeg, wv3]
    if last:
        out_specs = [bspec(_S), bspec(3 * _V)]
        out_shape = [jax.ShapeDtypeStruct((_N, _S), jnp.float32),
                     jax.ShapeDtypeStruct((_N, 3 * _V), jnp.float32)]
    else:
        in_specs += [wspec(_S, _S), wspec(1, _S), wspec(_S, _S), wspec(1, _S),
                     wspec(1, _S), wspec(1, _S)]
        args += [wu1, bu1.reshape(1, _S), wu2, bu2.reshape(1, _S),
                 gnext.reshape(1, _S), bnext.reshape(1, _S)]
        out_specs = bspec(_W)
        out_shape = jax.ShapeDtypeStruct((_N, _W), jnp.float32)
    return pl.pallas_call(
        functools.partial(_node_upd_body, last),
        grid=grid,
        in_specs=in_specs,
        out_specs=out_specs,
        out_shape=out_shape,
    )(*args)


def _build_buckets(dst):
    e_iota = jnp.arange(_E, dtype=jnp.int32)
    bucket = dst // _BKT
    sb, perm = lax.sort_key_val(bucket, e_iota)
    qs5 = jnp.arange(_NB + 1, dtype=jnp.int32)
    off_c = jnp.searchsorted(sb, qs5, side='left').astype(jnp.int32)
    cnt = off_c[1:] - off_c[:-1]
    cnt_pad = ((cnt + _CPB - 1) // _CPB) * _CPB
    off_pad = jnp.concatenate([jnp.zeros((1,), jnp.int32), jnp.cumsum(cnt_pad).astype(jnp.int32)])
    qs = jnp.arange(_EL, dtype=jnp.int32)
    bq = jnp.sum(qs[:, None] >= off_pad[None, 1:_NB], axis=1).astype(jnp.int32)
    rank = qs - off_pad[bq]
    valid = rank < cnt[bq]
    srci = jnp.clip(off_c[bq] + rank, 0, _E - 1)
    eids = jnp.where(valid, perm[srci], qs % _E)
    dstp = dst[perm]
    ldst = jnp.where(valid, dstp[srci] - _BKT * bq, _BKT + (qs % (_ACC - _BKT)))
    offs = jnp.zeros((16,), jnp.int32).at[:_NB + 1].set(off_pad)
    return eids, ldst, offs


def _forward(P, s, v, d, a, edge_index):
    n = s.shape[0]
    src = edge_index[0]
    dst = edge_index[1]
    eids, ldst, offs = _build_buckets(dst)
    zrow = jnp.zeros((_GCH, _W), jnp.float32)
    padi = (jnp.arange(_EPAD - _E, dtype=jnp.int32) % _N)
    src_p = jnp.concatenate([src, padi])
    dst_p = jnp.concatenate([dst, padi])
    zpad1 = jnp.zeros((_EPAD - _E,), jnp.float32)
    d2 = jnp.concatenate([d, zpad1])[:, None]
    a0 = jnp.concatenate([a[:, 0], zpad1])[:, None]
    a1 = jnp.concatenate([a[:, 1], zpad1])[:, None]
    a2 = jnp.concatenate([a[:, 2], zpad1])[:, None]

    vflat = v.reshape(n, 3 * _V)
    rdeg = None
    tab = _node_pre(s, vflat, P['gamma0'], P['beta0'])
    for l in range(_L):
        gd, gs = _sc_gather(tab, src_p, dst_p)
        pay = _edge_mlp(l, gd, gs, d2, a0, a1, a2,
                        P['Wrbf%d' % l], P['W1_%d' % l], P['b1_%d' % l],
                        P['W2_%d' % l], P['b2_%d' % l])
        agg = _sc_scatter(pay, eids, ldst, offs, zrow)
        agg = agg.reshape(_NB, _ACC, _W)[:, :_BKT, :].reshape(n, _W)
        if l == 0:
            rdeg = 1.0 / jnp.maximum(agg[:, _S + 3 * _V:_S + 3 * _V + 1], 1.0)
        wv3 = jnp.kron(jnp.eye(3, dtype=jnp.float32), P['Wv%d' % l])
        if l < _L - 1:
            tab = _node_upd(tab, agg, rdeg, wv3,
                            P['Wu1_%d' % l], P['bu1_%d' % l],
                            P['Wu2_%d' % l], P['bu2_%d' % l],
                            P['gamma%d' % (l + 1)], P['beta%d' % (l + 1)])
        else:
            s_out, v_out = _node_upd(tab, agg, rdeg, wv3,
                                     None, None, None, None, None, None)
    return s_out, v_out.reshape(n, 3, _V)


def kernel(s, v, edge_index, edge_d, edge_vec, gamma0, beta0, Wrbf0, W1_0, b1_0, W2_0, b2_0, Wv0, Wu1_0, bu1_0, Wu2_0, bu2_0, gamma1, beta1, Wrbf1, W1_1, b1_1, W2_1, b2_1, Wv1, Wu1_1, bu1_1, Wu2_1, bu2_1, gamma2, beta2, Wrbf2, W1_2, b1_2, W2_2, b2_2, Wv2, Wu1_2, bu1_2, Wu2_2, bu2_2, gamma3, beta3, Wrbf3, W1_3, b1_3, W2_3, b2_3, Wv3, Wu1_3, bu1_3, Wu2_3, bu2_3, gamma4, beta4, Wrbf4, W1_4, b1_4, W2_4, b2_4, Wv4):
    kw = dict(locals())
    edge_index = kw.pop('edge_index')
    s = kw.pop('s')
    v = kw.pop('v')
    d = kw.pop('edge_d')
    a = kw.pop('edge_vec')
    return _forward(kw, s, v, d, a, edge_index)


# speedup vs baseline: 23.6429x; 1.0420x over previous
import functools
import jax
import jax.numpy as jnp
import numpy as np
from jax import lax
from jax.experimental import pallas as pl
from jax.experimental.pallas import tpu as pltpu
from jax.experimental.pallas import tpu_sc as plsc

_S = 64
_V = 16
_R = 64
_CUTOFF = 10.0
_L = 5
_N = 50000
_E = 800000

_W = 128
_EB = 2048
_EPAD = 819200
_GW = 32
_GCH = 128
_GNCH = _EPAD // (_GW * _GCH)

_NB = 8
_BKT = 6250
_ACC = 6400
_PT = _ACC // 16
_CPB = 2048
_EL = _E + _NB * _CPB


def _edge_body(l, *refs):
    (gd_ref, gs_ref, d_ref, a0_ref, a1_ref, a2_ref,
     wrbf_ref, w1_ref, b1_ref, w2_ref, b2_ref, out_ref) = refs
    d = d_ref[:, 0:1]
    centers = jax.lax.broadcasted_iota(jnp.int32, (1, _R), 1).astype(jnp.float32) * (_CUTOFF / (_R - 1))
    width = _CUTOFF / _R
    rbf = jnp.exp(-0.5 * ((d - centers) / width) ** 2)
    env = 0.5 * (jnp.cos(jnp.pi * jnp.clip(d, 0.0, _CUTOFF) / _CUTOFF) + 1.0)
    filt = jnp.dot(rbf, wrbf_ref[...], preferred_element_type=jnp.float32)
    sd = gd_ref[:, 0:_S]
    ss = gs_ref[:, 0:_S]
    a0 = a0_ref[...]
    a1 = a1_ref[...]
    a2 = a2_ref[...]
    if l > 0:
        vs = gs_ref[:, _S:_S + 3 * _V]
        vdot = vs[:, 0:16] * a0 + vs[:, 16:32] * a1 + vs[:, 32:48] * a2
        m_in = jnp.concatenate([sd, ss, vdot], axis=1)
    else:
        m_in = jnp.concatenate([sd, ss], axis=1)
    h = m_in @ w1_ref[...] + b1_ref[...]
    h = h * jax.nn.sigmoid(h)
    h = h * filt
    o = h @ w2_ref[...] + b2_ref[...]
    ms = o[:, :_S] * env
    g0 = o[:, _S:_S + _V]
    g1 = o[:, _S + _V:]
    if l > 0:
        vm = jnp.concatenate([g0 * a0 + g1 * vs[:, 0:16],
                              g0 * a1 + g1 * vs[:, 16:32],
                              g0 * a2 + g1 * vs[:, 32:48]], axis=1)
    else:
        vm = jnp.concatenate([g0 * a0, g0 * a1, g0 * a2], axis=1)
    ones = jnp.ones((_EB, 1), jnp.float32)
    pad = jnp.zeros((_EB, _W - _S - 3 * _V - 1), jnp.float32)
    out_ref[...] = jnp.concatenate([ms, vm * env, ones, pad], axis=1)


def _edge_mlp(l, gd, gs, d2, a0, a1, a2, wrbf, w1, b1, w2, b2):
    grid = (_EPAD // _EB,)
    din = 2 * _S + (_V if l > 0 else 0)
    bspec = lambda w: pl.BlockSpec((_EB, w), lambda i: (i, 0))
    wspec = lambda r, c: pl.BlockSpec((r, c), lambda i: (0, 0))
    in_specs = [bspec(_W), bspec(_W), bspec(1), bspec(1), bspec(1), bspec(1),
                wspec(_R, _S), wspec(din, _S), wspec(1, _S),
                wspec(_S, _S + 2 * _V), wspec(1, _S + 2 * _V)]
    args = [gd, gs, d2, a0, a1, a2, wrbf, w1, b1.reshape(1, _S), w2, b2.reshape(1, _S + 2 * _V)]
    return pl.pallas_call(
        functools.partial(_edge_body, l),
        grid=grid,
        in_specs=in_specs,
        out_specs=pl.BlockSpec((_EB, _W), lambda i: (i, 0)),
        out_shape=jax.ShapeDtypeStruct((_EPAD, _W), jnp.float32),
    )(*args)


def _sc_mesh():
    return plsc.VectorSubcoreMesh(core_axis_name="c", subcore_axis_name="s")


def _gather_body(tab, srci, dsti, gd, gs,
                 ix0s, ix0d, ix1s, ix1d, bd0, bs0, bd1, bs1, sg0, sg1, so0, so1):
    wid = lax.axis_index("s") * 2 + lax.axis_index("c")
    base = wid * (_GNCH * _GCH)
    IX = ((ix0s, ix0d), (ix1s, ix1d))
    BF = ((bd0, bs0), (bd1, bs1))
    SG = (sg0, sg1)
    SO = (so0, so1)

    def load_idx(j, b):
        st = base + j * _GCH
        pltpu.sync_copy(srci.at[pl.ds(st, _GCH)], IX[b][0])
        pltpu.sync_copy(dsti.at[pl.ds(st, _GCH)], IX[b][1])

    def fire_g(b):
        pltpu.async_copy(tab.at[IX[b][1]], BF[b][0], SG[b])
        pltpu.async_copy(tab.at[IX[b][0]], BF[b][1], SG[b])

    def wait_g(b):
        pltpu.make_async_copy(tab.at[IX[b][1]], BF[b][0], SG[b]).wait()
        pltpu.make_async_copy(tab.at[IX[b][0]], BF[b][1], SG[b]).wait()

    def fire_o(j, b):
        st = base + j * _GCH
        pltpu.async_copy(BF[b][0], gd.at[pl.ds(st, _GCH)], SO[b])
        pltpu.async_copy(BF[b][1], gs.at[pl.ds(st, _GCH)], SO[b])

    def wait_o(j, b):
        st = base + j * _GCH
        pltpu.make_async_copy(BF[b][0], gd.at[pl.ds(st, _GCH)], SO[b]).wait()
        pltpu.make_async_copy(BF[b][1], gs.at[pl.ds(st, _GCH)], SO[b]).wait()

    load_idx(0, 0)
    fire_g(0)
    load_idx(1, 1)
    fire_g(1)

    def body(i, carry):
        for b in range(2):
            j = 2 * i + b
            wait_g(b)
            fire_o(j, b)
        for b in range(2):
            jn = 2 * i + 2 + b
            wait_o(jn - 2, b)
            load_idx(jn, b)
            fire_g(b)
        return carry

    lax.fori_loop(0, _GNCH // 2 - 1, body, 0)
    for b in range(2):
        wait_g(b)
        fire_o(_GNCH - 2 + b, b)
    for b in range(2):
        wait_o(_GNCH - 2 + b, b)


def _sc_gather(tab, srci, dsti):
    return pl.kernel(
        _gather_body,
        out_type=[jax.ShapeDtypeStruct((_EPAD, _W), jnp.float32),
                  jax.ShapeDtypeStruct((_EPAD, _W), jnp.float32)],
        mesh=_sc_mesh(),
        compiler_params=pltpu.CompilerParams(use_tc_tiling_on_sc=True),
        scratch_types=[
            pltpu.VMEM((_GCH,), jnp.int32),
            pltpu.VMEM((_GCH,), jnp.int32),
            pltpu.VMEM((_GCH,), jnp.int32),
            pltpu.VMEM((_GCH,), jnp.int32),
            pltpu.VMEM((_GCH, _W), jnp.float32),
            pltpu.VMEM((_GCH, _W), jnp.float32),
            pltpu.VMEM((_GCH, _W), jnp.float32),
            pltpu.VMEM((_GCH, _W), jnp.float32),
            pltpu.SemaphoreType.DMA,
            pltpu.SemaphoreType.DMA,
            pltpu.SemaphoreType.DMA,
            pltpu.SemaphoreType.DMA,
        ],
    )(tab, srci, dsti)


def _scatter_body(pay, eids, ldst, offs, zrow, out,
                  offv, eidb0, eidb1, ldb0, ldb1, pbuf0, pbuf1, zbuf, acc, sem0, sem1):
    c = lax.axis_index("c")
    s = lax.axis_index("s")
    pltpu.sync_copy(offs, offv)
    pltpu.sync_copy(zrow, zbuf)
    ov = offv[...]
    oly = [ov[i] for i in range(_NB + 1)]
    EID = (eidb0, eidb1)
    LDB = (ldb0, ldb1)
    PBF = (pbuf0, pbuf1)
    SEM = (sem0, sem1)

    nfull = _PT // _GCH
    rem = _PT % _GCH
    for bb in range(_NB // 2):
        b = c * (_NB // 2) + bb
        off_b = lax.select(c == 0, oly[bb], oly[_NB // 2 + bb])
        off_b1 = lax.select(c == 0, oly[bb + 1], oly[_NB // 2 + bb + 1])
        nch = (off_b1 - off_b) // _GCH
        nj = (nch + 15 - s) // 16
        for k in range(nfull):
            pltpu.sync_copy(zbuf, acc.at[pl.ds(s * _PT + k * _GCH, _GCH)])
        if rem:
            pltpu.sync_copy(zbuf.at[pl.ds(0, rem)],
                            acc.at[pl.ds(s * _PT + nfull * _GCH, rem)])
        plsc.subcore_barrier()

        def chunk_load(j, u):
            st = pl.multiple_of(off_b + (j * 16 + s) * _GCH, _GCH)
            pltpu.sync_copy(eids.at[pl.ds(st, _GCH)], EID[u])
            pltpu.sync_copy(ldst.at[pl.ds(st, _GCH)], LDB[u].at[0])
            pltpu.async_copy(pay.at[EID[u]], PBF[u], SEM[u])

        def chunk_add(u):
            pltpu.make_async_copy(pay.at[EID[u]], PBF[u], SEM[u]).wait()
            pltpu.sync_copy(PBF[u], acc.at[LDB[u].at[0]], add=True)

        def body(i, carry):
            for u in range(2):
                chunk_load(2 * i + u, u)
            for u in range(2):
                chunk_add(u)
            return carry

        lax.fori_loop(0, nj // 2, body, 0)

        @pl.when(nj % 2 == 1)
        def _tail():
            chunk_load(nj - 1, 0)
            chunk_add(0)

        plsc.subcore_barrier()
        ob = b * _ACC + s * _PT
        for k in range(nfull):
            pltpu.sync_copy(acc.at[pl.ds(s * _PT + k * _GCH, _GCH)], pbuf0)
            pltpu.sync_copy(pbuf0, out.at[pl.ds(ob + k * _GCH, _GCH)])
        if rem:
            pltpu.sync_copy(acc.at[pl.ds(s * _PT + nfull * _GCH, rem)],
                            pbuf0.at[pl.ds(0, rem)])
            pltpu.sync_copy(pbuf0.at[pl.ds(0, rem)],
                            out.at[pl.ds(ob + nfull * _GCH, rem)])
        plsc.subcore_barrier()


def _sc_scatter(pay, eids, ldst, offs, zrow):
    return pl.kernel(
        _scatter_body,
        out_type=jax.ShapeDtypeStruct((_NB * _ACC, _W), jnp.float32),
        mesh=_sc_mesh(),
        compiler_params=pltpu.CompilerParams(use_tc_tiling_on_sc=True),
        scratch_types=[
            pltpu.VMEM((16,), jnp.int32),
            pltpu.VMEM((_GCH,), jnp.int32),
            pltpu.VMEM((_GCH,), jnp.int32),
            pltpu.VMEM((1, _GCH), jnp.int32),
            pltpu.VMEM((1, _GCH), jnp.int32),
            pltpu.VMEM((_GCH, _W), jnp.float32),
            pltpu.VMEM((_GCH, _W), jnp.float32),
            pltpu.VMEM((_GCH, _W), jnp.float32),
            pltpu.VMEM_SHARED((_ACC, _W), jnp.float32),
            pltpu.SemaphoreType.DMA,
            pltpu.SemaphoreType.DMA,
        ],
    )(pay, eids, ldst, offs, zrow)


_NBLK = 2000


def _norm_tab(s1, v1, g_ref, b_ref):
    mu = jnp.mean(s1, axis=-1, keepdims=True)
    var = jnp.mean((s1 - mu) ** 2, axis=-1, keepdims=True)
    sn = (s1 - mu) / jnp.sqrt(var + 1e-6) * g_ref[...] + b_ref[...]
    vn = jnp.sqrt(jnp.mean(v1 * v1, axis=1, keepdims=True) + 1e-6)
    vnf = v1 / vn
    pad = jnp.zeros((s1.shape[0], _W - _S - 3 * _V), jnp.float32)
    return jnp.concatenate([sn, vnf, pad], axis=1)


def _node_pre_body(s_ref, v_ref, g_ref, b_ref, tab_ref):
    tab_ref[...] = _norm_tab(s_ref[...], v_ref[...], g_ref, b_ref)


def _node_pre(s, vflat, gamma, beta):
    grid = (_N // _NBLK,)
    return pl.pallas_call(
        _node_pre_body,
        grid=grid,
        in_specs=[pl.BlockSpec((_NBLK, _S), lambda i: (i, 0)),
                  pl.BlockSpec((_NBLK, 3 * _V), lambda i: (i, 0)),
                  pl.BlockSpec((1, _S), lambda i: (0, 0)),
                  pl.BlockSpec((1, _S), lambda i: (0, 0))],
        out_specs=pl.BlockSpec((_NBLK, _W), lambda i: (i, 0)),
        out_shape=jax.ShapeDtypeStruct((_N, _W), jnp.float32),
    )(s, vflat, gamma.reshape(1, _S), beta.reshape(1, _S))


def _node_upd_body(last, *refs):
    if last:
        (tab_ref, agg_ref, rdeg_ref, wv3_ref, s_out, v_out) = refs
    else:
        (tab_ref, agg_ref, rdeg_ref, wv3_ref,
         wu1_ref, bu1_ref, wu2_ref, bu2_ref, g_ref, b_ref, tab_out) = refs
    tab = tab_ref[...]
    agg = agg_ref[...]
    rdeg = rdeg_ref[...]
    s1 = tab[:, 0:_S] + agg[:, 0:_S]
    va = agg[:, _S:_S + 3 * _V] * rdeg
    v1 = tab[:, _S:_S + 3 * _V] + jnp.dot(va, wv3_ref[...], preferred_element_type=jnp.float32)
    if last:
        s_out[...] = s1
        v_out[...] = v1
    else:
        h = s1 @ wu1_ref[...] + bu1_ref[...]
        h = h * jax.nn.sigmoid(h)
        s1 = s1 + h @ wu2_ref[...] + bu2_ref[...]
        tab_out[...] = _norm_tab(s1, v1, g_ref, b_ref)


def _node_upd(tab, agg, rdeg, wv3, wu1, bu1, wu2, bu2, gnext, bnext):
    last = wu1 is None
    grid = (_N // _NBLK,)
    bspec = lambda w: pl.BlockSpec((_NBLK, w), lambda i: (i, 0))
    wspec = lambda r, c: pl.BlockSpec((r, c), lambda i: (0, 0))
    in_specs = [bspec(_W), bspec(_W), bspec(1), wspec(3 * _V, 3 * _V)]
    args = [tab, agg, rdeg, wv3]
    if last:
        out_specs = [bspec(_S), bspec(3 * _V)]
        out_shape = [jax.ShapeDtypeStruct((_N, _S), jnp.float32),
                     jax.ShapeDtypeStruct((_N, 3 * _V), jnp.float32)]
    else:
        in_specs += [wspec(_S, _S), wspec(1, _S), wspec(_S, _S), wspec(1, _S),
                     wspec(1, _S), wspec(1, _S)]
        args += [wu1, bu1.reshape(1, _S), wu2, bu2.reshape(1, _S),
                 gnext.reshape(1, _S), bnext.reshape(1, _S)]
        out_specs = bspec(_W)
        out_shape = jax.ShapeDtypeStruct((_N, _W), jnp.float32)
    return pl.pallas_call(
        functools.partial(_node_upd_body, last),
        grid=grid,
        in_specs=in_specs,
        out_specs=out_specs,
        out_shape=out_shape,
    )(*args)


def _build_buckets(dst):
    e_iota = jnp.arange(_E, dtype=jnp.int32)
    bucket = dst // _BKT
    sb, perm = lax.sort_key_val(bucket, e_iota)
    qs5 = jnp.arange(_NB + 1, dtype=jnp.int32)
    off_c = jnp.searchsorted(sb, qs5, side='left').astype(jnp.int32)
    cnt = off_c[1:] - off_c[:-1]
    cnt_pad = ((cnt + _CPB - 1) // _CPB) * _CPB
    off_pad = jnp.concatenate([jnp.zeros((1,), jnp.int32), jnp.cumsum(cnt_pad).astype(jnp.int32)])
    qs = jnp.arange(_EL, dtype=jnp.int32)
    bq = jnp.sum(qs[:, None] >= off_pad[None, 1:_NB], axis=1).astype(jnp.int32)
    rank = qs - off_pad[bq]
    valid = rank < cnt[bq]
    srci = jnp.clip(off_c[bq] + rank, 0, _E - 1)
    eids = jnp.where(valid, perm[srci], qs % _E)
    dstp = dst[perm]
    ldst = jnp.where(valid, dstp[srci] - _BKT * bq, _BKT + (qs % (_ACC - _BKT)))
    offs = jnp.zeros((16,), jnp.int32).at[:_NB + 1].set(off_pad)
    return eids, ldst, offs


def _forward(P, s, v, d, a, edge_index):
    n = s.shape[0]
    src = edge_index[0]
    dst = edge_index[1]
    eids, ldst, offs = _build_buckets(dst)
    zrow = jnp.zeros((_GCH, _W), jnp.float32)
    padi = (jnp.arange(_EPAD - _E, dtype=jnp.int32) % _N)
    src_p = jnp.concatenate([src, padi])
    dst_p = jnp.concatenate([dst, padi])
    zpad1 = jnp.zeros((_EPAD - _E,), jnp.float32)
    d2 = jnp.concatenate([d, zpad1])[:, None]
    a0 = jnp.concatenate([a[:, 0], zpad1])[:, None]
    a1 = jnp.concatenate([a[:, 1], zpad1])[:, None]
    a2 = jnp.concatenate([a[:, 2], zpad1])[:, None]

    vflat = v.reshape(n, 3 * _V)
    rdeg = None
    tab = _node_pre(s, vflat, P['gamma0'], P['beta0'])
    for l in range(_L):
        gd, gs = _sc_gather(tab, src_p, dst_p)
        pay = _edge_mlp(l, gd, gs, d2, a0, a1, a2,
                        P['Wrbf%d' % l], P['W1_%d' % l], P['b1_%d' % l],
                        P['W2_%d' % l], P['b2_%d' % l])
        agg = _sc_scatter(pay, eids, ldst, offs, zrow)
        agg = agg.reshape(_NB, _ACC, _W)[:, :_BKT, :].reshape(n, _W)
        if l == 0:
            rdeg = 1.0 / jnp.maximum(agg[:, _S + 3 * _V:_S + 3 * _V + 1], 1.0)
        wv3 = jnp.kron(jnp.eye(3, dtype=jnp.float32), P['Wv%d' % l])
        if l < _L - 1:
            tab = _node_upd(tab, agg, rdeg, wv3,
                            P['Wu1_%d' % l], P['bu1_%d' % l],
                            P['Wu2_%d' % l], P['bu2_%d' % l],
                            P['gamma%d' % (l + 1)], P['beta%d' % (l + 1)])
        else:
            s_out, v_out = _node_upd(tab, agg, rdeg, wv3,
                                     None, None, None, None, None, None)
    return s_out, v_out.reshape(n, 3, _V)


def kernel(s, v, edge_index, edge_d, edge_vec, gamma0, beta0, Wrbf0, W1_0, b1_0, W2_0, b2_0, Wv0, Wu1_0, bu1_0, Wu2_0, bu2_0, gamma1, beta1, Wrbf1, W1_1, b1_1, W2_1, b2_1, Wv1, Wu1_1, bu1_1, Wu2_1, bu2_1, gamma2, beta2, Wrbf2, W1_2, b1_2, W2_2, b2_2, Wv2, Wu1_2, bu1_2, Wu2_2, bu2_2, gamma3, beta3, Wrbf3, W1_3, b1_3, W2_3, b2_3, Wv3, Wu1_3, bu1_3, Wu2_3, bu2_3, gamma4, beta4, Wrbf4, W1_4, b1_4, W2_4, b2_4, Wv4):
    kw = dict(locals())
    edge_index = kw.pop('edge_index')
    s = kw.pop('s')
    v = kw.pop('v')
    d = kw.pop('edge_d')
    a = kw.pop('edge_vec')
    return _forward(kw, s, v, d, a, edge_index)
